# baseline (speedup 1.0000x reference)
import functools

import jax
import jax.numpy as jnp
from jax import lax
from jax.experimental import pallas as pl
from jax.experimental.pallas import tpu as pltpu

N = 10000
E = 160000
F_IN = 23
H1, C1 = 8, 120
H2, C2 = 1, 1


def _proj_kernel(x_ref, wl_ref, wr_ref, bl_ref, br_ref, xl_ref, xr_ref):
    x = x_ref[...]
    xl_ref[...] = jnp.dot(x, wl_ref[...], preferred_element_type=jnp.float32) + bl_ref[...]
    xr_ref[...] = jnp.dot(x, wr_ref[...], preferred_element_type=jnp.float32) + br_ref[...]


def _proj(x, Wl, bl, Wr, br):
    n, f = x.shape
    k = Wl.shape[1]
    blk = 1000
    grid = (n // blk,)
    return pl.pallas_call(
        _proj_kernel,
        grid=grid,
        in_specs=[
            pl.BlockSpec((blk, f), lambda i: (i, 0)),
            pl.BlockSpec((f, k), lambda i: (0, 0)),
            pl.BlockSpec((f, k), lambda i: (0, 0)),
            pl.BlockSpec((k,), lambda i: (0,)),
            pl.BlockSpec((k,), lambda i: (0,)),
        ],
        out_specs=[
            pl.BlockSpec((blk, k), lambda i: (i, 0)),
            pl.BlockSpec((blk, k), lambda i: (i, 0)),
        ],
        out_shape=[
            jax.ShapeDtypeStruct((n, k), jnp.float32),
            jax.ShapeDtypeStruct((n, k), jnp.float32),
        ],
    )(x, Wl, Wr, bl, br)


def _gatv2_rest(xl, xr, src, dst, att, bias, heads, ch):
    xl = xl.reshape(-1, heads, ch)
    xr = xr.reshape(-1, heads, ch)
    e = xl[src] + xr[dst]
    e = jax.nn.leaky_relu(e, negative_slope=0.2)
    alpha = jnp.sum(e * att, axis=-1)
    amax = jax.ops.segment_max(alpha, dst, num_segments=N)
    amax = jnp.where(jnp.isfinite(amax), amax, 0.0)
    ealpha = jnp.exp(alpha - amax[dst])
    denom = jax.ops.segment_sum(ealpha, dst, num_segments=N)
    coef = ealpha / (denom[dst] + 1e-16)
    msg = xl[src] * coef[:, :, None]
    out = jax.ops.segment_sum(msg, dst, num_segments=N)
    return out.reshape(N, heads * ch) + bias


def kernel(x, edge_index, Wl1, bl1, Wr1, br1, att1, bias1, Wl2, bl2, Wr2, br2, att2, bias2):
    src = edge_index[0]
    dst = edge_index[1]
    xl1, xr1 = _proj(x, Wl1, bl1, Wr1, br1)
    h = _gatv2_rest(xl1, xr1, src, dst, att1, bias1, H1, C1)
    h = jax.nn.relu(h)
    xl2 = h @ Wl2 + bl2
    xr2 = h @ Wr2 + br2
    out = _gatv2_rest(xl2, xr2, src, dst, att2, bias2, H2, C2)
    return out


# speedup vs baseline: 3.7156x; 3.7156x over previous
import jax
import jax.numpy as jnp
from jax import lax
from jax.experimental import pallas as pl
from jax.experimental.pallas import tpu as pltpu
from jax.experimental.pallas import tpu_sc as plsc

N = 10000
E = 160000
H1, C1 = 8, 120
CP = 128
D = H1 * CP
W = 32
EPW = E // W
GB = 8
NB = EPW // GB
NPW = 320
NPT = W * NPW
CH = 2000
NCH = E // CH
RC = 125
NPC = N // RC
FL = 16

_MESH = dict(core_axis_name="c", subcore_axis_name="s")
_GDN = lax.GatherDimensionNumbers(
    offset_dims=(), collapsed_slice_dims=(0,), start_index_map=(0,))


def _lane_perm(x, idx):
    return lax.gather(x, idx[:, None], _GDN, (1,),
                      mode=lax.GatherScatterMode.PROMISE_IN_BOUNDS)


def _lane_iota():
    return lax.broadcasted_iota(jnp.int32, (16,), 0)


def _to_scalar(v):
    return jnp.where(_lane_iota() == 0, v, jnp.zeros_like(v))[0]


def _dyn_lane(v, i):
    return _to_scalar(_lane_perm(v, jnp.zeros((16,), jnp.int32) + i))


def _lane_count(mask):
    lane = _lane_iota()
    x = jnp.where(mask, jnp.ones((16,), jnp.int32), jnp.zeros((16,), jnp.int32))
    for sft in (1, 2, 4, 8):
        x = x + _lane_perm(x, (lane + sft) & 15)
    return x


def _lane_min(x):
    lane = _lane_iota()
    for sft in (1, 2, 4, 8):
        x = jnp.minimum(x, _lane_perm(x, (lane + sft) & 15))
    return x


def _wid():
    return lax.axis_index("s") * 2 + lax.axis_index("c")


def _proj_kernel(x_ref, wl_ref, wr_ref, bl_ref, br_ref, xl_ref, xr_ref):
    x = x_ref[...]
    xl_ref[...] = jnp.dot(x, wl_ref[...], preferred_element_type=jnp.float32) + bl_ref[...]
    xr_ref[...] = jnp.dot(x, wr_ref[...], preferred_element_type=jnp.float32) + br_ref[...]


def _proj(x, Wl, bl, Wr, br):
    n, f = x.shape
    k = Wl.shape[1]
    blk = 1000
    return pl.pallas_call(
        _proj_kernel,
        grid=(n // blk,),
        in_specs=[
            pl.BlockSpec((blk, f), lambda i: (i, 0)),
            pl.BlockSpec((f, k), lambda i: (0, 0)),
            pl.BlockSpec((f, k), lambda i: (0, 0)),
            pl.BlockSpec((k,), lambda i: (0,)),
            pl.BlockSpec((k,), lambda i: (0,)),
        ],
        out_specs=[
            pl.BlockSpec((blk, k), lambda i: (i, 0)),
            pl.BlockSpec((blk, k), lambda i: (i, 0)),
        ],
        out_shape=[
            jax.ShapeDtypeStruct((n, k), jnp.float32),
            jax.ShapeDtypeStruct((n, k), jnp.float32),
        ],
    )(x, Wl, Wr, bl, br)


def _pass_a_body(xl_hbm, xr_hbm, src_hbm, dst_hbm, att_hbm, alpha_hbm,
                 src_v, dst_v, att_v, lb0, lb1, rb0, rb1, a0, a1,
                 gs0, gs1, ws0, ws1):
    base = _wid() * EPW
    pltpu.sync_copy(src_hbm.at[pl.ds(base, EPW)], src_v)
    pltpu.sync_copy(dst_hbm.at[pl.ds(base, EPW)], dst_v)
    pltpu.sync_copy(att_hbm, att_v)
    zeros16 = jnp.zeros((16,), jnp.float32)
    lane = _lane_iota()
    rots = [(lane + s) & 15 for s in (1, 2, 4, 8)]

    def fire(k, lb, rb, gs):
        pltpu.async_copy(xl_hbm.at[src_v.at[pl.ds(k * GB, GB)]], lb, gs)
        pltpu.async_copy(xr_hbm.at[dst_v.at[pl.ds(k * GB, GB)]], rb, gs)

    def waitg(lb, rb, gs):
        pltpu.make_async_copy(xl_hbm.at[pl.ds(0, GB)], lb, gs).wait()
        pltpu.make_async_copy(xr_hbm.at[pl.ds(0, GB)], rb, gs).wait()

    def compute(k, lb, rb, ab, ws):
        @pl.when(k >= 2)
        def _():
            pltpu.make_async_copy(ab, alpha_hbm.at[pl.ds(base * 16, GB * 16)], ws).wait()

        def edge_body(e, _):
            row = zeros16
            for h in range(H1):
                acc = zeros16
                for v in range(CP // 16):
                    off = h * CP + v * 16
                    t = lb[e, pl.ds(off, 16)] + rb[e, pl.ds(off, 16)]
                    t = jnp.maximum(t, t * 0.2)
                    acc = acc + t * att_v[h, pl.ds(v * 16, 16)]
                for r in rots:
                    acc = acc + _lane_perm(acc, r)
                row = jnp.where(lane == h, acc, row)
            eo = pl.multiple_of(e * 16, 16)
            ab[pl.ds(eo, 16)] = row
            return 0

        lax.fori_loop(0, GB, edge_body, 0)
        pltpu.async_copy(ab, alpha_hbm.at[pl.ds((base + k * GB) * 16, GB * 16)], ws)

    fire(0, lb0, rb0, gs0)
    fire(1, lb1, rb1, gs1)

    def loop_body(bi, _):
        k0 = 2 * bi
        waitg(lb0, rb0, gs0)
        compute(k0, lb0, rb0, a0, ws0)

        @pl.when(k0 + 2 < NB)
        def _():
            fire(k0 + 2, lb0, rb0, gs0)

        waitg(lb1, rb1, gs1)
        compute(k0 + 1, lb1, rb1, a1, ws1)

        @pl.when(k0 + 3 < NB)
        def _():
            fire(k0 + 3, lb1, rb1, gs1)

        return 0

    lax.fori_loop(0, NB // 2, loop_body, 0)
    waitg(lb0, rb0, gs0)
    compute(NB - 1, lb0, rb0, a0, ws0)
    pltpu.make_async_copy(a0, alpha_hbm.at[pl.ds(base * 16, GB * 16)], ws0).wait()
    pltpu.make_async_copy(a1, alpha_hbm.at[pl.ds(base * 16, GB * 16)], ws1).wait()


def _pass_a(xlp, xrp, src, dst, attp):
    kfn = pl.kernel(
        _pass_a_body,
        out_type=jax.ShapeDtypeStruct((E * 16,), jnp.float32),
        mesh=plsc.VectorSubcoreMesh(**_MESH),
        scratch_types=[
            pltpu.VMEM((EPW,), jnp.int32),
            pltpu.VMEM((EPW,), jnp.int32),
            pltpu.VMEM((H1, CP), jnp.float32),
            pltpu.VMEM((GB, D), jnp.float32),
            pltpu.VMEM((GB, D), jnp.float32),
            pltpu.VMEM((GB, D), jnp.float32),
            pltpu.VMEM((GB, D), jnp.float32),
            pltpu.VMEM((GB * 16,), jnp.float32),
            pltpu.VMEM((GB * 16,), jnp.float32),
            pltpu.SemaphoreType.DMA,
            pltpu.SemaphoreType.DMA,
            pltpu.SemaphoreType.DMA,
            pltpu.SemaphoreType.DMA,
        ],
    )
    return kfn(xlp, xrp, src, dst, attp)


def _pass_b_body(dst_hbm, alpha_hbm, mtab_hbm, stab_hbm,
                 dv0, av0, dv1, av1, mtab, stab, cs0, cs1):
    lo = _wid() * NPW
    neg = jnp.full((16,), -1e30, jnp.float32)
    zeros16 = jnp.zeros((16,), jnp.float32)
    lane = _lane_iota()

    def init_body(i, _):
        o = pl.multiple_of(i * 16, 16)
        mtab[pl.ds(o, 16)] = neg
        stab[pl.ds(o, 16)] = zeros16
        return 0

    lax.fori_loop(0, NPW, init_body, 0)

    def fire(ci, dv, av, cs):
        pltpu.async_copy(dst_hbm.at[pl.ds(ci * CH, CH)], dv, cs)
        pltpu.async_copy(alpha_hbm.at[pl.ds(ci * CH * 16, CH * 16)], av, cs)

    def waitc(dv, av, cs):
        pltpu.make_async_copy(dst_hbm.at[pl.ds(0, CH)], dv, cs).wait()
        pltpu.make_async_copy(alpha_hbm.at[pl.ds(0, CH * 16)], av, cs).wait()

    def process(dv, av):
        def group(gi, _):
            gbase = pl.multiple_of(gi * 16, 16)
            d = dv[pl.ds(gbase, 16)]
            dl = d - lo
            mask = (dl >= 0) & (dl < NPW)
            tot = _lane_count(mask)
            lv0 = jnp.where(mask, lane, 16)

            def wbody(k, lv):
                i0 = _to_scalar(_lane_min(lv))
                dli = _dyn_lane(dl, i0)
                to = pl.multiple_of(dli * 16, 16)
                ao = pl.multiple_of((gbase + i0) * 16, 16)
                arow = av[pl.ds(ao, 16)]
                m0 = mtab[pl.ds(to, 16)]
                mn = jnp.maximum(m0, arow)
                em = jnp.exp(m0 - mn)
                stab[pl.ds(to, 16)] = stab[pl.ds(to, 16)] * em + jnp.exp(arow - mn)
                mtab[pl.ds(to, 16)] = mn
                return jnp.where(lane == i0, 16, lv)

            lax.fori_loop(0, _to_scalar(tot), wbody, lv0)
            return 0

        lax.fori_loop(0, CH // 16, group, 0)

    fire(0, dv0, av0, cs0)
    fire(1, dv1, av1, cs1)

    def chunk_loop(hi, _):
        c0 = 2 * hi
        waitc(dv0, av0, cs0)
        process(dv0, av0)

        @pl.when(c0 + 2 < NCH)
        def _():
            fire(c0 + 2, dv0, av0, cs0)

        waitc(dv1, av1, cs1)
        process(dv1, av1)

        @pl.when(c0 + 3 < NCH)
        def _():
            fire(c0 + 3, dv1, av1, cs1)

        return 0

    lax.fori_loop(0, NCH // 2, chunk_loop, 0)
    pltpu.sync_copy(mtab, mtab_hbm.at[pl.ds(lo * 16, NPW * 16)])
    pltpu.sync_copy(stab, stab_hbm.at[pl.ds(lo * 16, NPW * 16)])


def _pass_b(dst, alpha):
    kfn = pl.kernel(
        _pass_b_body,
        out_type=[
            jax.ShapeDtypeStruct((NPT * 16,), jnp.float32),
            jax.ShapeDtypeStruct((NPT * 16,), jnp.float32),
        ],
        mesh=plsc.VectorSubcoreMesh(**_MESH),
        scratch_types=[
            pltpu.VMEM((CH,), jnp.int32),
            pltpu.VMEM((CH * 16,), jnp.float32),
            pltpu.VMEM((CH,), jnp.int32),
            pltpu.VMEM((CH * 16,), jnp.float32),
            pltpu.VMEM((NPW * 16,), jnp.float32),
            pltpu.VMEM((NPW * 16,), jnp.float32),
            pltpu.SemaphoreType.DMA,
            pltpu.SemaphoreType.DMA,
        ],
    )
    return kfn(dst, alpha)


def _pass_c_body(xl_hbm, src_hbm, dst_hbm, alpha_hbm, mtab_hbm, stab_hbm,
                 bias_hbm, hout_hbm,
                 sv0, dv0, sv1, dv1, mtv, stv, acc, biasv,
                 idxs, idxw, xst, astw, cs0, cs1, gs0):
    wid = _wid()
    pltpu.sync_copy(bias_hbm, biasv)
    zeros16 = jnp.zeros((16,), jnp.float32)
    zi = jnp.zeros((16,), jnp.int32)
    lane = _lane_iota()

    def flush(cnt, pd, ps, pj):
        idxs[...] = ps

        def widx(e, _):
            o = pl.multiple_of(e * 16, 16)
            idxw[pl.ds(o, 16)] = _dyn_lane(pj, e) * 16 + lane
            return 0

        lax.fori_loop(0, FL, widx, 0)
        cx = pltpu.async_copy(xl_hbm.at[idxs], xst, gs0)
        ca = pltpu.async_copy(alpha_hbm.at[idxw], astw, gs0)
        cx.wait()
        ca.wait()

        def pe(e, _):
            dl_e = _dyn_lane(pd, e)
            to = pl.multiple_of(dl_e * 16, 16)
            ao = pl.multiple_of(e * 16, 16)
            arow = astw[pl.ds(ao, 16)]
            c = jnp.exp(arow - mtv[pl.ds(to, 16)]) / (stv[pl.ds(to, 16)] + 1e-16)
            for h in range(H1):
                chs = c[h]
                for v in range(CP // 16):
                    off = h * CP + v * 16
                    acc[dl_e, pl.ds(off, 16)] = (
                        acc[dl_e, pl.ds(off, 16)] + xst[e, pl.ds(off, 16)] * chs)
            return 0

        lax.fori_loop(0, cnt, pe, 0)

    def range_pass(rp, _):
        rng = rp * W + wid

        @pl.when(rng < RC)
        def _():
            lo = rng * NPC
            pltpu.sync_copy(mtab_hbm.at[pl.ds(lo * 16, NPC * 16)], mtv)
            pltpu.sync_copy(stab_hbm.at[pl.ds(lo * 16, NPC * 16)], stv)

            def z(i, _):
                for v in range(D // 16):
                    acc[i, pl.ds(v * 16, 16)] = zeros16
                return 0

            lax.fori_loop(0, NPC, z, 0)

            def do_chunk(c0, sv, dv, carry):
                def group(gi, carry):
                    pd, ps, pj, pcv = carry
                    gbase = pl.multiple_of(gi * 16, 16)
                    d = dv[pl.ds(gbase, 16)]
                    sg = sv[pl.ds(gbase, 16)]
                    dl = d - lo
                    mask = (dl >= 0) & (dl < NPC)
                    tot = _lane_count(mask)
                    tot_s = _to_scalar(tot)
                    lv0 = jnp.where(mask, lane, 16)

                    def compact(k, st4):
                        hd, hs, hj, lv = st4
                        i0 = _to_scalar(_lane_min(lv))
                        hd = jnp.where(lane == k, _dyn_lane(dl, i0), hd)
                        hs = jnp.where(lane == k, _dyn_lane(sg, i0), hs)
                        hj = jnp.where(lane == k, c0 * CH + gbase + i0, hj)
                        lv = jnp.where(lane == i0, 16, lv)
                        return (hd, hs, hj, lv)

                    hd, hs, hj, _lv = lax.fori_loop(
                        0, tot_s, compact, (zi, zi, zi, lv0))
                    pc = _to_scalar(pcv)
                    shd = _lane_perm(hd, (lane - pc) & 15)
                    shs = _lane_perm(hs, (lane - pc) & 15)
                    shj = _lane_perm(hj, (lane - pc) & 15)
                    pd_n = jnp.where(lane >= pc, shd, pd)
                    ps_n = jnp.where(lane >= pc, shs, ps)
                    pj_n = jnp.where(lane >= pc, shj, pj)
                    m = pc + tot_s

                    @pl.when(m >= FL)
                    def _():
                        flush(FL, pd_n, ps_n, pj_n)

                    pd_a = _lane_perm(hd, (lane + (FL - pc)) & 15)
                    ps_a = _lane_perm(hs, (lane + (FL - pc)) & 15)
                    pj_a = _lane_perm(hj, (lane + (FL - pc)) & 15)
                    ovf = m >= FL
                    pd_c = jnp.where(ovf, pd_a, pd_n)
                    ps_c = jnp.where(ovf, ps_a, ps_n)
                    pj_c = jnp.where(ovf, pj_a, pj_n)
                    pcv_c = zi + jnp.where(ovf, m - FL, m)
                    return (pd_c, ps_c, pj_c, pcv_c)

                return lax.fori_loop(0, CH // 16, group, carry)

            def fire(ci, sv, dv, cs):
                pltpu.async_copy(src_hbm.at[pl.ds(ci * CH, CH)], sv, cs)
                pltpu.async_copy(dst_hbm.at[pl.ds(ci * CH, CH)], dv, cs)

            def waitc(sv, dv, cs):
                pltpu.make_async_copy(src_hbm.at[pl.ds(0, CH)], sv, cs).wait()
                pltpu.make_async_copy(dst_hbm.at[pl.ds(0, CH)], dv, cs).wait()

            fire(0, sv0, dv0, cs0)
            fire(1, sv1, dv1, cs1)

            def chunk_loop(hi, carry):
                c0 = 2 * hi
                waitc(sv0, dv0, cs0)
                carry = do_chunk(c0, sv0, dv0, carry)

                @pl.when(c0 + 2 < NCH)
                def _():
                    fire(c0 + 2, sv0, dv0, cs0)

                waitc(sv1, dv1, cs1)
                carry = do_chunk(c0 + 1, sv1, dv1, carry)

                @pl.when(c0 + 3 < NCH)
                def _():
                    fire(c0 + 3, sv1, dv1, cs1)

                return carry

            carry0 = (zi, zi, zi, zi)
            pd, ps, pj, pcv = lax.fori_loop(0, NCH // 2, chunk_loop, carry0)
            pc = _to_scalar(pcv)

            @pl.when(pc > 0)
            def _():
                flush(pc, pd, ps, pj)

            def ep(i, _):
                for v in range(D // 16):
                    o = v * 16
                    acc[i, pl.ds(o, 16)] = jnp.maximum(
                        acc[i, pl.ds(o, 16)] + biasv[pl.ds(o, 16)], 0.0)
                return 0

            lax.fori_loop(0, NPC, ep, 0)
            pltpu.sync_copy(acc, hout_hbm.at[pl.ds(lo, NPC)])

        return 0

    lax.fori_loop(0, (RC + W - 1) // W, range_pass, 0)


def _pass_c(xlp, src, dst, alpha, mtab, stab, biasp):
    kfn = pl.kernel(
        _pass_c_body,
        out_type=jax.ShapeDtypeStruct((N, D), jnp.float32),
        mesh=plsc.VectorSubcoreMesh(**_MESH),
        scratch_types=[
            pltpu.VMEM((CH,), jnp.int32),
            pltpu.VMEM((CH,), jnp.int32),
            pltpu.VMEM((CH,), jnp.int32),
            pltpu.VMEM((CH,), jnp.int32),
            pltpu.VMEM((NPC * 16,), jnp.float32),
            pltpu.VMEM((NPC * 16,), jnp.float32),
            pltpu.VMEM((NPC, D), jnp.float32),
            pltpu.VMEM((D,), jnp.float32),
            pltpu.VMEM((FL,), jnp.int32),
            pltpu.VMEM((FL * 16,), jnp.int32),
            pltpu.VMEM((FL, D), jnp.float32),
            pltpu.VMEM((FL * 16,), jnp.float32),
            pltpu.SemaphoreType.DMA,
            pltpu.SemaphoreType.DMA,
            pltpu.SemaphoreType.DMA,
        ],
    )
    return kfn(xlp, src, dst, alpha, mtab, stab, biasp)


def _pass_d_body(src_hbm, dst_hbm, xl2_hbm, xr2_hbm, sc2_hbm, out_hbm,
                 sv0, dv0, sv1, dv1, xl2v, xr2v, mt, st, vt, sc2v, cs0, cs1):
    lo = _wid() * NPW
    neg = jnp.full((16,), -1e30, jnp.float32)
    zeros16 = jnp.zeros((16,), jnp.float32)
    lane = _lane_iota()
    pltpu.sync_copy(xl2_hbm, xl2v)
    pltpu.sync_copy(xr2_hbm, xr2v)
    pltpu.sync_copy(sc2_hbm, sc2v)
    scv = sc2v[pl.ds(0, 16)]
    att2s = scv[0]
    bias2s = scv[1]

    def init_body(i, _):
        o = pl.multiple_of(i * 16, 16)
        mt[pl.ds(o, 16)] = neg
        st[pl.ds(o, 16)] = zeros16
        vt[pl.ds(o, 16)] = zeros16
        return 0

    lax.fori_loop(0, NPW, init_body, 0)

    def fire(ci, sv, dv, cs):
        pltpu.async_copy(src_hbm.at[pl.ds(ci * CH, CH)], sv, cs)
        pltpu.async_copy(dst_hbm.at[pl.ds(ci * CH, CH)], dv, cs)

    def waitc(sv, dv, cs):
        pltpu.make_async_copy(src_hbm.at[pl.ds(0, CH)], sv, cs).wait()
        pltpu.make_async_copy(dst_hbm.at[pl.ds(0, CH)], dv, cs).wait()

    def tab_read(tab, i):
        ib = pl.multiple_of((i >> 4) << 4, 16)
        return _dyn_lane(tab[pl.ds(ib, 16)], i & 15)

    def process(sv, dv):
        def group(gi, _):
            gbase = pl.multiple_of(gi * 16, 16)
            d = dv[pl.ds(gbase, 16)]
            sg = sv[pl.ds(gbase, 16)]
            dl = d - lo
            mask = (dl >= 0) & (dl < NPW)
            tot = _lane_count(mask)
            lv0 = jnp.where(mask, lane, 16)

            def wbody(k, lv):
                i0 = _to_scalar(_lane_min(lv))
                dli = _dyn_lane(dl, i0)
                si = _dyn_lane(sg, i0)
                di = _dyn_lane(d, i0)
                xls = tab_read(xl2v, si)
                t = xls + tab_read(xr2v, di)
                a2 = att2s * jnp.maximum(t, 0.2 * t)
                to = pl.multiple_of(dli * 16, 16)
                m0 = mt[pl.ds(to, 16)]
                mn = jnp.maximum(m0, a2)
                em = jnp.exp(m0 - mn)
                ea = jnp.exp(a2 - mn)
                st[pl.ds(to, 16)] = st[pl.ds(to, 16)] * em + ea
                vt[pl.ds(to, 16)] = vt[pl.ds(to, 16)] * em + ea * xls
                mt[pl.ds(to, 16)] = mn
                return jnp.where(lane == i0, 16, lv)

            lax.fori_loop(0, _to_scalar(tot), wbody, lv0)
            return 0

        lax.fori_loop(0, CH // 16, group, 0)

    fire(0, sv0, dv0, cs0)
    fire(1, sv1, dv1, cs1)

    def chunk_loop(hi, _):
        c0 = 2 * hi
        waitc(sv0, dv0, cs0)
        process(sv0, dv0)

        @pl.when(c0 + 2 < NCH)
        def _():
            fire(c0 + 2, sv0, dv0, cs0)

        waitc(sv1, dv1, cs1)
        process(sv1, dv1)

        @pl.when(c0 + 3 < NCH)
        def _():
            fire(c0 + 3, sv1, dv1, cs1)

        return 0

    lax.fori_loop(0, NCH // 2, chunk_loop, 0)

    def ep(i, _):
        o = pl.multiple_of(i * 16, 16)
        vt[pl.ds(o, 16)] = vt[pl.ds(o, 16)] / (st[pl.ds(o, 16)] + 1e-16) + bias2s
        return 0

    lax.fori_loop(0, NPW, ep, 0)
    pltpu.sync_copy(vt, out_hbm.at[pl.ds(lo * 16, NPW * 16)])


def _pass_d(src, dst, xl2, xr2, sc2):
    kfn = pl.kernel(
        _pass_d_body,
        out_type=jax.ShapeDtypeStruct((NPT * 16,), jnp.float32),
        mesh=plsc.VectorSubcoreMesh(**_MESH),
        scratch_types=[
            pltpu.VMEM((CH,), jnp.int32),
            pltpu.VMEM((CH,), jnp.int32),
            pltpu.VMEM((CH,), jnp.int32),
            pltpu.VMEM((CH,), jnp.int32),
            pltpu.VMEM((NPT,), jnp.float32),
            pltpu.VMEM((NPT,), jnp.float32),
            pltpu.VMEM((NPW * 16,), jnp.float32),
            pltpu.VMEM((NPW * 16,), jnp.float32),
            pltpu.VMEM((NPW * 16,), jnp.float32),
            pltpu.VMEM((16,), jnp.float32),
            pltpu.SemaphoreType.DMA,
            pltpu.SemaphoreType.DMA,
        ],
    )
    return kfn(src, dst, xl2, xr2, sc2)


def _pad_heads(w, heads, ch):
    lead = w.shape[:-1]
    w = w.reshape(lead + (heads, ch))
    w = jnp.pad(w, [(0, 0)] * len(lead) + [(0, 0), (0, CP - ch)])
    return w.reshape(lead + (heads * CP,))


def kernel(x, edge_index, Wl1, bl1, Wr1, br1, att1, bias1, Wl2, bl2, Wr2, br2, att2, bias2):
    src = edge_index[0]
    dst = edge_index[1]

    Wl1p = _pad_heads(Wl1, H1, C1)
    Wr1p = _pad_heads(Wr1, H1, C1)
    bl1p = _pad_heads(bl1, H1, C1)
    br1p = _pad_heads(br1, H1, C1)
    attp = _pad_heads(att1.reshape(1, H1 * C1), H1, C1).reshape(H1, CP)
    bias1p = _pad_heads(bias1, H1, C1)

    xlp, xrp = _proj(x, Wl1p, bl1p, Wr1p, br1p)
    alpha = _pass_a(xlp, xrp, src, dst, attp)
    mtab, stab = _pass_b(dst, alpha)
    hp = _pass_c(xlp, src, dst, alpha, mtab, stab, bias1p)

    Wl2p = _pad_heads(Wl2.T, H1, C1).T
    Wr2p = _pad_heads(Wr2.T, H1, C1).T
    xl2, xr2 = _proj(hp, Wl2p, bl2, Wr2p, br2)
    xl2p = jnp.pad(xl2[:, 0], (0, NPT - N))
    xr2p = jnp.pad(xr2[:, 0], (0, NPT - N))
    sc2 = jnp.concatenate([att2.reshape(1), bias2.reshape(1),
                           jnp.zeros((14,), jnp.float32)])
    out2 = _pass_d(src, dst, xl2p, xr2p, sc2)
    return out2.reshape(NPT, 16)[:N, 0:1]


# speedup vs baseline: 4.0207x; 1.0821x over previous
import jax
import jax.numpy as jnp
from jax import lax
from jax.experimental import pallas as pl
from jax.experimental.pallas import tpu as pltpu
from jax.experimental.pallas import tpu_sc as plsc

N = 10000
E = 160000
H1, C1 = 8, 120
CP = 128
D = H1 * CP
W = 32
EPW = E // W
GB = 8
NB = EPW // GB
NPW = 320
NPT = W * NPW
CH = 2000
NCH = E // CH
RC = 125
NPC = N // RC
FL = 16

_MESH = dict(core_axis_name="c", subcore_axis_name="s")
_GDN = lax.GatherDimensionNumbers(
    offset_dims=(), collapsed_slice_dims=(0,), start_index_map=(0,))


def _lane_perm(x, idx):
    return lax.gather(x, idx[:, None], _GDN, (1,),
                      mode=lax.GatherScatterMode.PROMISE_IN_BOUNDS)


def _lane_iota():
    return lax.broadcasted_iota(jnp.int32, (16,), 0)


def _to_scalar(v):
    return jnp.where(_lane_iota() == 0, v, jnp.zeros_like(v))[0]


def _dyn_lane(v, i):
    return _to_scalar(_lane_perm(v, jnp.zeros((16,), jnp.int32) + i))


def _lane_count(mask):
    lane = _lane_iota()
    x = jnp.where(mask, jnp.ones((16,), jnp.int32), jnp.zeros((16,), jnp.int32))
    for sft in (1, 2, 4, 8):
        x = x + _lane_perm(x, (lane + sft) & 15)
    return x


def _lane_min(x):
    lane = _lane_iota()
    for sft in (1, 2, 4, 8):
        x = jnp.minimum(x, _lane_perm(x, (lane + sft) & 15))
    return x


def _wid():
    return lax.axis_index("s") * 2 + lax.axis_index("c")


def _proj_kernel(x_ref, wl_ref, wr_ref, bl_ref, br_ref, xl_ref, xr_ref):
    x = x_ref[...]
    xl_ref[...] = jnp.dot(x, wl_ref[...], preferred_element_type=jnp.float32) + bl_ref[...]
    xr_ref[...] = jnp.dot(x, wr_ref[...], preferred_element_type=jnp.float32) + br_ref[...]


def _proj(x, Wl, bl, Wr, br):
    n, f = x.shape
    k = Wl.shape[1]
    blk = 1000
    return pl.pallas_call(
        _proj_kernel,
        grid=(n // blk,),
        in_specs=[
            pl.BlockSpec((blk, f), lambda i: (i, 0)),
            pl.BlockSpec((f, k), lambda i: (0, 0)),
            pl.BlockSpec((f, k), lambda i: (0, 0)),
            pl.BlockSpec((k,), lambda i: (0,)),
            pl.BlockSpec((k,), lambda i: (0,)),
        ],
        out_specs=[
            pl.BlockSpec((blk, k), lambda i: (i, 0)),
            pl.BlockSpec((blk, k), lambda i: (i, 0)),
        ],
        out_shape=[
            jax.ShapeDtypeStruct((n, k), jnp.float32),
            jax.ShapeDtypeStruct((n, k), jnp.float32),
        ],
    )(x, Wl, Wr, bl, br)


def _pass_a_body(xl_hbm, xr_hbm, src_hbm, dst_hbm, att_hbm, alpha_hbm,
                 src_v, dst_v, att_v, lb0, lb1, rb0, rb1, a0, a1,
                 gs0, gs1, ws0, ws1):
    base = _wid() * EPW
    pltpu.sync_copy(src_hbm.at[pl.ds(base, EPW)], src_v)
    pltpu.sync_copy(dst_hbm.at[pl.ds(base, EPW)], dst_v)
    pltpu.sync_copy(att_hbm, att_v)
    zeros16 = jnp.zeros((16,), jnp.float32)
    lane = _lane_iota()
    rots = [(lane + s) & 15 for s in (1, 2, 4, 8)]

    def fire(k, lb, rb, gs):
        pltpu.async_copy(xl_hbm.at[src_v.at[pl.ds(k * GB, GB)]], lb, gs)
        pltpu.async_copy(xr_hbm.at[dst_v.at[pl.ds(k * GB, GB)]], rb, gs)

    def waitg(lb, rb, gs):
        pltpu.make_async_copy(xl_hbm.at[pl.ds(0, GB)], lb, gs).wait()
        pltpu.make_async_copy(xr_hbm.at[pl.ds(0, GB)], rb, gs).wait()

    def compute(k, lb, rb, ab, ws):
        @pl.when(k >= 2)
        def _():
            pltpu.make_async_copy(ab, alpha_hbm.at[pl.ds(base * 16, GB * 16)], ws).wait()

        def edge_body(e, _):
            row = zeros16
            for h in range(H1):
                acc = zeros16
                for v in range(CP // 16):
                    off = h * CP + v * 16
                    t = lb[e, pl.ds(off, 16)] + rb[e, pl.ds(off, 16)]
                    t = jnp.maximum(t, t * 0.2)
                    acc = acc + t * att_v[h, pl.ds(v * 16, 16)]
                for r in rots:
                    acc = acc + _lane_perm(acc, r)
                row = jnp.where(lane == h, acc, row)
            eo = pl.multiple_of(e * 16, 16)
            ab[pl.ds(eo, 16)] = row
            return 0

        lax.fori_loop(0, GB, edge_body, 0)
        pltpu.async_copy(ab, alpha_hbm.at[pl.ds((base + k * GB) * 16, GB * 16)], ws)

    fire(0, lb0, rb0, gs0)
    fire(1, lb1, rb1, gs1)

    def loop_body(bi, _):
        k0 = 2 * bi
        waitg(lb0, rb0, gs0)
        compute(k0, lb0, rb0, a0, ws0)

        @pl.when(k0 + 2 < NB)
        def _():
            fire(k0 + 2, lb0, rb0, gs0)

        waitg(lb1, rb1, gs1)
        compute(k0 + 1, lb1, rb1, a1, ws1)

        @pl.when(k0 + 3 < NB)
        def _():
            fire(k0 + 3, lb1, rb1, gs1)

        return 0

    lax.fori_loop(0, NB // 2, loop_body, 0)
    waitg(lb0, rb0, gs0)
    compute(NB - 1, lb0, rb0, a0, ws0)
    pltpu.make_async_copy(a0, alpha_hbm.at[pl.ds(base * 16, GB * 16)], ws0).wait()
    pltpu.make_async_copy(a1, alpha_hbm.at[pl.ds(base * 16, GB * 16)], ws1).wait()


def _pass_a(xlp, xrp, src, dst, attp):
    kfn = pl.kernel(
        _pass_a_body,
        out_type=jax.ShapeDtypeStruct((E * 16,), jnp.float32),
        mesh=plsc.VectorSubcoreMesh(**_MESH),
        scratch_types=[
            pltpu.VMEM((EPW,), jnp.int32),
            pltpu.VMEM((EPW,), jnp.int32),
            pltpu.VMEM((H1, CP), jnp.float32),
            pltpu.VMEM((GB, D), jnp.float32),
            pltpu.VMEM((GB, D), jnp.float32),
            pltpu.VMEM((GB, D), jnp.float32),
            pltpu.VMEM((GB, D), jnp.float32),
            pltpu.VMEM((GB * 16,), jnp.float32),
            pltpu.VMEM((GB * 16,), jnp.float32),
            pltpu.SemaphoreType.DMA,
            pltpu.SemaphoreType.DMA,
            pltpu.SemaphoreType.DMA,
            pltpu.SemaphoreType.DMA,
        ],
    )
    return kfn(xlp, xrp, src, dst, attp)


def _pass_b_body(dst_hbm, alpha_hbm, mtab_hbm, stab_hbm,
                 dv0, av0, dv1, av1, mtab, stab, cs0, cs1):
    lo = _wid() * NPW
    neg = jnp.full((16,), -1e30, jnp.float32)
    zeros16 = jnp.zeros((16,), jnp.float32)
    lane = _lane_iota()

    def init_body(i, _):
        o = pl.multiple_of(i * 16, 16)
        mtab[pl.ds(o, 16)] = neg
        stab[pl.ds(o, 16)] = zeros16
        return 0

    lax.fori_loop(0, NPW, init_body, 0)

    def fire(ci, dv, av, cs):
        pltpu.async_copy(dst_hbm.at[pl.ds(ci * CH, CH)], dv, cs)
        pltpu.async_copy(alpha_hbm.at[pl.ds(ci * CH * 16, CH * 16)], av, cs)

    def waitc(dv, av, cs):
        pltpu.make_async_copy(dst_hbm.at[pl.ds(0, CH)], dv, cs).wait()
        pltpu.make_async_copy(alpha_hbm.at[pl.ds(0, CH * 16)], av, cs).wait()

    def process(dv, av):
        def group(gi, _):
            gbase = pl.multiple_of(gi * 16, 16)
            d = dv[pl.ds(gbase, 16)]
            dl = d - lo
            mask = (dl >= 0) & (dl < NPW)
            tot = _lane_count(mask)
            lv0 = jnp.where(mask, lane, 16)

            def wbody(k, lv):
                i0 = _to_scalar(_lane_min(lv))
                dli = _dyn_lane(dl, i0)
                to = pl.multiple_of(dli * 16, 16)
                ao = pl.multiple_of((gbase + i0) * 16, 16)
                arow = av[pl.ds(ao, 16)]
                m0 = mtab[pl.ds(to, 16)]
                mn = jnp.maximum(m0, arow)
                em = jnp.exp(m0 - mn)
                stab[pl.ds(to, 16)] = stab[pl.ds(to, 16)] * em + jnp.exp(arow - mn)
                mtab[pl.ds(to, 16)] = mn
                return jnp.where(lane == i0, 16, lv)

            lax.fori_loop(0, _to_scalar(tot), wbody, lv0)
            return 0

        lax.fori_loop(0, CH // 16, group, 0)

    fire(0, dv0, av0, cs0)
    fire(1, dv1, av1, cs1)

    def chunk_loop(hi, _):
        c0 = 2 * hi
        waitc(dv0, av0, cs0)
        process(dv0, av0)

        @pl.when(c0 + 2 < NCH)
        def _():
            fire(c0 + 2, dv0, av0, cs0)

        waitc(dv1, av1, cs1)
        process(dv1, av1)

        @pl.when(c0 + 3 < NCH)
        def _():
            fire(c0 + 3, dv1, av1, cs1)

        return 0

    lax.fori_loop(0, NCH // 2, chunk_loop, 0)
    pltpu.sync_copy(mtab, mtab_hbm.at[pl.ds(lo * 16, NPW * 16)])
    pltpu.sync_copy(stab, stab_hbm.at[pl.ds(lo * 16, NPW * 16)])


def _pass_b(dst, alpha):
    kfn = pl.kernel(
        _pass_b_body,
        out_type=[
            jax.ShapeDtypeStruct((NPT * 16,), jnp.float32),
            jax.ShapeDtypeStruct((NPT * 16,), jnp.float32),
        ],
        mesh=plsc.VectorSubcoreMesh(**_MESH),
        scratch_types=[
            pltpu.VMEM((CH,), jnp.int32),
            pltpu.VMEM((CH * 16,), jnp.float32),
            pltpu.VMEM((CH,), jnp.int32),
            pltpu.VMEM((CH * 16,), jnp.float32),
            pltpu.VMEM((NPW * 16,), jnp.float32),
            pltpu.VMEM((NPW * 16,), jnp.float32),
            pltpu.SemaphoreType.DMA,
            pltpu.SemaphoreType.DMA,
        ],
    )
    return kfn(dst, alpha)


def _pass_c_body(xl_hbm, src_hbm, dst_hbm, alpha_hbm, mtab_hbm, stab_hbm,
                 bias_hbm, hout_hbm,
                 sv0, dv0, sv1, dv1, mtv, stv, acc, biasv,
                 idxs, idxw, xst, astw, pdref, psref, pjref, pcref,
                 cs0, cs1, gs0):
    wid = _wid()
    pltpu.sync_copy(bias_hbm, biasv)
    zeros16 = jnp.zeros((16,), jnp.float32)
    zi = jnp.zeros((16,), jnp.int32)
    lane = _lane_iota()

    def flush(cnt, pd, ps, pj):
        idxs[...] = ps

        def widx(e, _):
            o = pl.multiple_of(e * 16, 16)
            idxw[pl.ds(o, 16)] = _dyn_lane(pj, e) * 16 + lane
            return 0

        lax.fori_loop(0, FL, widx, 0)
        cx = pltpu.async_copy(xl_hbm.at[idxs], xst, gs0)
        ca = pltpu.async_copy(alpha_hbm.at[idxw], astw, gs0)
        cx.wait()
        ca.wait()

        def pe(e, _):
            dl_e = _dyn_lane(pd, e)
            to = pl.multiple_of(dl_e * 16, 16)
            ao = pl.multiple_of(e * 16, 16)
            arow = astw[pl.ds(ao, 16)]
            c = jnp.exp(arow - mtv[pl.ds(to, 16)]) / (stv[pl.ds(to, 16)] + 1e-16)
            for h in range(H1):
                chs = c[h]
                for v in range(CP // 16):
                    off = h * CP + v * 16
                    acc[dl_e, pl.ds(off, 16)] = (
                        acc[dl_e, pl.ds(off, 16)] + xst[e, pl.ds(off, 16)] * chs)
            return 0

        lax.fori_loop(0, cnt, pe, 0)

    def range_pass(rp, _):
        rng = rp * W + wid

        @pl.when(rng < RC)
        def _():
            lo = rng * NPC
            pltpu.sync_copy(mtab_hbm.at[pl.ds(lo * 16, NPC * 16)], mtv)
            pltpu.sync_copy(stab_hbm.at[pl.ds(lo * 16, NPC * 16)], stv)

            def z(i, _):
                for v in range(D // 16):
                    acc[i, pl.ds(v * 16, 16)] = zeros16
                return 0

            lax.fori_loop(0, NPC, z, 0)

            def do_chunk(c0, sv, dv):
                def group(gi, _):
                    gbase = pl.multiple_of(gi * 16, 16)
                    d = dv[pl.ds(gbase, 16)]
                    sg = sv[pl.ds(gbase, 16)]
                    dl = d - lo
                    mask = (dl >= 0) & (dl < NPC)
                    tot = _lane_count(mask)
                    tot_s = _to_scalar(tot)

                    @pl.when(tot_s > 0)
                    def _():
                        pd = pdref[pl.ds(0, 16)]
                        ps = psref[pl.ds(0, 16)]
                        pj = pjref[pl.ds(0, 16)]
                        pcv = pcref[pl.ds(0, 16)]
                        lv0 = jnp.where(mask, lane, 16)

                        def compact(k, st4):
                            hd, hs, hj, lv = st4
                            i0 = _to_scalar(_lane_min(lv))
                            hd = jnp.where(lane == k, _dyn_lane(dl, i0), hd)
                            hs = jnp.where(lane == k, _dyn_lane(sg, i0), hs)
                            hj = jnp.where(lane == k, c0 * CH + gbase + i0, hj)
                            lv = jnp.where(lane == i0, 16, lv)
                            return (hd, hs, hj, lv)

                        hd, hs, hj, _lv = lax.fori_loop(
                            0, tot_s, compact, (zi, zi, zi, lv0))
                        pc = _to_scalar(pcv)
                        shd = _lane_perm(hd, (lane - pc) & 15)
                        shs = _lane_perm(hs, (lane - pc) & 15)
                        shj = _lane_perm(hj, (lane - pc) & 15)
                        pd_n = jnp.where(lane >= pc, shd, pd)
                        ps_n = jnp.where(lane >= pc, shs, ps)
                        pj_n = jnp.where(lane >= pc, shj, pj)
                        m = pc + tot_s

                        @pl.when(m >= FL)
                        def _():
                            flush(FL, pd_n, ps_n, pj_n)

                        pd_a = _lane_perm(hd, (lane + (FL - pc)) & 15)
                        ps_a = _lane_perm(hs, (lane + (FL - pc)) & 15)
                        pj_a = _lane_perm(hj, (lane + (FL - pc)) & 15)
                        ovf = m >= FL
                        pdref[...] = jnp.where(ovf, pd_a, pd_n)
                        psref[...] = jnp.where(ovf, ps_a, ps_n)
                        pjref[...] = jnp.where(ovf, pj_a, pj_n)
                        pcref[...] = zi + jnp.where(ovf, m - FL, m)

                    return 0

                lax.fori_loop(0, CH // 16, group, 0)

            def fire(ci, sv, dv, cs):
                pltpu.async_copy(src_hbm.at[pl.ds(ci * CH, CH)], sv, cs)
                pltpu.async_copy(dst_hbm.at[pl.ds(ci * CH, CH)], dv, cs)

            def waitc(sv, dv, cs):
                pltpu.make_async_copy(src_hbm.at[pl.ds(0, CH)], sv, cs).wait()
                pltpu.make_async_copy(dst_hbm.at[pl.ds(0, CH)], dv, cs).wait()

            pdref[...] = zi
            psref[...] = zi
            pjref[...] = zi
            pcref[...] = zi
            fire(0, sv0, dv0, cs0)
            fire(1, sv1, dv1, cs1)

            def chunk_loop(hi, _):
                c0 = 2 * hi
                waitc(sv0, dv0, cs0)
                do_chunk(c0, sv0, dv0)

                @pl.when(c0 + 2 < NCH)
                def _():
                    fire(c0 + 2, sv0, dv0, cs0)

                waitc(sv1, dv1, cs1)
                do_chunk(c0 + 1, sv1, dv1)

                @pl.when(c0 + 3 < NCH)
                def _():
                    fire(c0 + 3, sv1, dv1, cs1)

                return 0

            lax.fori_loop(0, NCH // 2, chunk_loop, 0)
            pc = _to_scalar(pcref[pl.ds(0, 16)])

            @pl.when(pc > 0)
            def _():
                flush(pc, pdref[pl.ds(0, 16)], psref[pl.ds(0, 16)],
                      pjref[pl.ds(0, 16)])

            def ep(i, _):
                for v in range(D // 16):
                    o = v * 16
                    acc[i, pl.ds(o, 16)] = jnp.maximum(
                        acc[i, pl.ds(o, 16)] + biasv[pl.ds(o, 16)], 0.0)
                return 0

            lax.fori_loop(0, NPC, ep, 0)
            pltpu.sync_copy(acc, hout_hbm.at[pl.ds(lo, NPC)])

        return 0

    lax.fori_loop(0, (RC + W - 1) // W, range_pass, 0)


def _pass_c(xlp, src, dst, alpha, mtab, stab, biasp):
    kfn = pl.kernel(
        _pass_c_body,
        out_type=jax.ShapeDtypeStruct((N, D), jnp.float32),
        mesh=plsc.VectorSubcoreMesh(**_MESH),
        scratch_types=[
            pltpu.VMEM((CH,), jnp.int32),
            pltpu.VMEM((CH,), jnp.int32),
            pltpu.VMEM((CH,), jnp.int32),
            pltpu.VMEM((CH,), jnp.int32),
            pltpu.VMEM((NPC * 16,), jnp.float32),
            pltpu.VMEM((NPC * 16,), jnp.float32),
            pltpu.VMEM((NPC, D), jnp.float32),
            pltpu.VMEM((D,), jnp.float32),
            pltpu.VMEM((FL,), jnp.int32),
            pltpu.VMEM((FL * 16,), jnp.int32),
            pltpu.VMEM((FL, D), jnp.float32),
            pltpu.VMEM((FL * 16,), jnp.float32),
            pltpu.VMEM((16,), jnp.int32),
            pltpu.VMEM((16,), jnp.int32),
            pltpu.VMEM((16,), jnp.int32),
            pltpu.VMEM((16,), jnp.int32),
            pltpu.SemaphoreType.DMA,
            pltpu.SemaphoreType.DMA,
            pltpu.SemaphoreType.DMA,
        ],
    )
    return kfn(xlp, src, dst, alpha, mtab, stab, biasp)


def _pass_d_body(src_hbm, dst_hbm, xl2_hbm, xr2_hbm, sc2_hbm, out_hbm,
                 sv0, dv0, sv1, dv1, xl2v, xr2v, mt, st, vt, sc2v, cs0, cs1):
    lo = _wid() * NPW
    neg = jnp.full((16,), -1e30, jnp.float32)
    zeros16 = jnp.zeros((16,), jnp.float32)
    lane = _lane_iota()
    pltpu.sync_copy(xl2_hbm, xl2v)
    pltpu.sync_copy(xr2_hbm, xr2v)
    pltpu.sync_copy(sc2_hbm, sc2v)
    scv = sc2v[pl.ds(0, 16)]
    att2s = scv[0]
    bias2s = scv[1]

    def init_body(i, _):
        o = pl.multiple_of(i * 16, 16)
        mt[pl.ds(o, 16)] = neg
        st[pl.ds(o, 16)] = zeros16
        vt[pl.ds(o, 16)] = zeros16
        return 0

    lax.fori_loop(0, NPW, init_body, 0)

    def fire(ci, sv, dv, cs):
        pltpu.async_copy(src_hbm.at[pl.ds(ci * CH, CH)], sv, cs)
        pltpu.async_copy(dst_hbm.at[pl.ds(ci * CH, CH)], dv, cs)

    def waitc(sv, dv, cs):
        pltpu.make_async_copy(src_hbm.at[pl.ds(0, CH)], sv, cs).wait()
        pltpu.make_async_copy(dst_hbm.at[pl.ds(0, CH)], dv, cs).wait()

    def tab_read(tab, i):
        ib = pl.multiple_of((i >> 4) << 4, 16)
        return _dyn_lane(tab[pl.ds(ib, 16)], i & 15)

    def process(sv, dv):
        def group(gi, _):
            gbase = pl.multiple_of(gi * 16, 16)
            d = dv[pl.ds(gbase, 16)]
            sg = sv[pl.ds(gbase, 16)]
            dl = d - lo
            mask = (dl >= 0) & (dl < NPW)
            tot = _lane_count(mask)
            lv0 = jnp.where(mask, lane, 16)

            def wbody(k, lv):
                i0 = _to_scalar(_lane_min(lv))
                dli = _dyn_lane(dl, i0)
                si = _dyn_lane(sg, i0)
                di = _dyn_lane(d, i0)
                xls = tab_read(xl2v, si)
                t = xls + tab_read(xr2v, di)
                a2 = att2s * jnp.maximum(t, 0.2 * t)
                to = pl.multiple_of(dli * 16, 16)
                m0 = mt[pl.ds(to, 16)]
                mn = jnp.maximum(m0, a2)
                em = jnp.exp(m0 - mn)
                ea = jnp.exp(a2 - mn)
                st[pl.ds(to, 16)] = st[pl.ds(to, 16)] * em + ea
                vt[pl.ds(to, 16)] = vt[pl.ds(to, 16)] * em + ea * xls
                mt[pl.ds(to, 16)] = mn
                return jnp.where(lane == i0, 16, lv)

            lax.fori_loop(0, _to_scalar(tot), wbody, lv0)
            return 0

        lax.fori_loop(0, CH // 16, group, 0)

    fire(0, sv0, dv0, cs0)
    fire(1, sv1, dv1, cs1)

    def chunk_loop(hi, _):
        c0 = 2 * hi
        waitc(sv0, dv0, cs0)
        process(sv0, dv0)

        @pl.when(c0 + 2 < NCH)
        def _():
            fire(c0 + 2, sv0, dv0, cs0)

        waitc(sv1, dv1, cs1)
        process(sv1, dv1)

        @pl.when(c0 + 3 < NCH)
        def _():
            fire(c0 + 3, sv1, dv1, cs1)

        return 0

    lax.fori_loop(0, NCH // 2, chunk_loop, 0)

    def ep(i, _):
        o = pl.multiple_of(i * 16, 16)
        vt[pl.ds(o, 16)] = vt[pl.ds(o, 16)] / (st[pl.ds(o, 16)] + 1e-16) + bias2s
        return 0

    lax.fori_loop(0, NPW, ep, 0)
    pltpu.sync_copy(vt, out_hbm.at[pl.ds(lo * 16, NPW * 16)])


def _pass_d(src, dst, xl2, xr2, sc2):
    kfn = pl.kernel(
        _pass_d_body,
        out_type=jax.ShapeDtypeStruct((NPT * 16,), jnp.float32),
        mesh=plsc.VectorSubcoreMesh(**_MESH),
        scratch_types=[
            pltpu.VMEM((CH,), jnp.int32),
            pltpu.VMEM((CH,), jnp.int32),
            pltpu.VMEM((CH,), jnp.int32),
            pltpu.VMEM((CH,), jnp.int32),
            pltpu.VMEM((NPT,), jnp.float32),
            pltpu.VMEM((NPT,), jnp.float32),
            pltpu.VMEM((NPW * 16,), jnp.float32),
            pltpu.VMEM((NPW * 16,), jnp.float32),
            pltpu.VMEM((NPW * 16,), jnp.float32),
            pltpu.VMEM((16,), jnp.float32),
            pltpu.SemaphoreType.DMA,
            pltpu.SemaphoreType.DMA,
        ],
    )
    return kfn(src, dst, xl2, xr2, sc2)


def _pad_heads(w, heads, ch):
    lead = w.shape[:-1]
    w = w.reshape(lead + (heads, ch))
    w = jnp.pad(w, [(0, 0)] * len(lead) + [(0, 0), (0, CP - ch)])
    return w.reshape(lead + (heads * CP,))


def kernel(x, edge_index, Wl1, bl1, Wr1, br1, att1, bias1, Wl2, bl2, Wr2, br2, att2, bias2):
    src = edge_index[0]
    dst = edge_index[1]

    Wl1p = _pad_heads(Wl1, H1, C1)
    Wr1p = _pad_heads(Wr1, H1, C1)
    bl1p = _pad_heads(bl1, H1, C1)
    br1p = _pad_heads(br1, H1, C1)
    attp = _pad_heads(att1.reshape(1, H1 * C1), H1, C1).reshape(H1, CP)
    bias1p = _pad_heads(bias1, H1, C1)

    xlp, xrp = _proj(x, Wl1p, bl1p, Wr1p, br1p)
    alpha = _pass_a(xlp, xrp, src, dst, attp)
    mtab, stab = _pass_b(dst, alpha)
    hp = _pass_c(xlp, src, dst, alpha, mtab, stab, bias1p)

    Wl2p = _pad_heads(Wl2.T, H1, C1).T
    Wr2p = _pad_heads(Wr2.T, H1, C1).T
    xl2, xr2 = _proj(hp, Wl2p, bl2, Wr2p, br2)
    xl2p = jnp.pad(xl2[:, 0], (0, NPT - N))
    xr2p = jnp.pad(xr2[:, 0], (0, NPT - N))
    sc2 = jnp.concatenate([att2.reshape(1), bias2.reshape(1),
                           jnp.zeros((14,), jnp.float32)])
    out2 = _pass_d(src, dst, xl2p, xr2p, sc2)
    return out2.reshape(NPT, 16)[:N, 0:1]


# speedup vs baseline: 4.0332x; 1.0031x over previous
import jax
import jax.numpy as jnp
from jax import lax
from jax.experimental import pallas as pl
from jax.experimental.pallas import tpu as pltpu
from jax.experimental.pallas import tpu_sc as plsc

N = 10000
E = 160000
H1, C1 = 8, 120
CP = 128
D = H1 * CP
W = 32
EPW = E // W
GB = 8
NB = EPW // GB
NPW = 320
NPT = W * NPW
CH = 2000
NCH = E // CH
RC = 125
NPC = N // RC
FL = 16

_MESH = dict(core_axis_name="c", subcore_axis_name="s")
_GDN = lax.GatherDimensionNumbers(
    offset_dims=(), collapsed_slice_dims=(0,), start_index_map=(0,))


def _lane_perm(x, idx):
    return lax.gather(x, idx[:, None], _GDN, (1,),
                      mode=lax.GatherScatterMode.PROMISE_IN_BOUNDS)


def _lane_iota():
    return lax.broadcasted_iota(jnp.int32, (16,), 0)


def _to_scalar(v):
    return jnp.where(_lane_iota() == 0, v, jnp.zeros_like(v))[0]


def _dyn_lane(v, i):
    return _to_scalar(_lane_perm(v, jnp.zeros((16,), jnp.int32) + i))


def _lane_count(mask):
    lane = _lane_iota()
    x = jnp.where(mask, jnp.ones((16,), jnp.int32), jnp.zeros((16,), jnp.int32))
    for sft in (1, 2, 4, 8):
        x = x + _lane_perm(x, (lane + sft) & 15)
    return x


def _lane_min(x):
    lane = _lane_iota()
    for sft in (1, 2, 4, 8):
        x = jnp.minimum(x, _lane_perm(x, (lane + sft) & 15))
    return x


def _wid():
    return lax.axis_index("s") * 2 + lax.axis_index("c")


def _proj_kernel(x_ref, wl_ref, wr_ref, bl_ref, br_ref, xl_ref, xr_ref):
    x = x_ref[...]
    xl_ref[...] = jnp.dot(x, wl_ref[...], preferred_element_type=jnp.float32) + bl_ref[...]
    xr_ref[...] = jnp.dot(x, wr_ref[...], preferred_element_type=jnp.float32) + br_ref[...]


def _proj(x, Wl, bl, Wr, br):
    n, f = x.shape
    k = Wl.shape[1]
    blk = 1000
    return pl.pallas_call(
        _proj_kernel,
        grid=(n // blk,),
        in_specs=[
            pl.BlockSpec((blk, f), lambda i: (i, 0)),
            pl.BlockSpec((f, k), lambda i: (0, 0)),
            pl.BlockSpec((f, k), lambda i: (0, 0)),
            pl.BlockSpec((k,), lambda i: (0,)),
            pl.BlockSpec((k,), lambda i: (0,)),
        ],
        out_specs=[
            pl.BlockSpec((blk, k), lambda i: (i, 0)),
            pl.BlockSpec((blk, k), lambda i: (i, 0)),
        ],
        out_shape=[
            jax.ShapeDtypeStruct((n, k), jnp.float32),
            jax.ShapeDtypeStruct((n, k), jnp.float32),
        ],
    )(x, Wl, Wr, bl, br)


def _pass_a_body(xl_hbm, xr_hbm, src_hbm, dst_hbm, att_hbm, alpha_hbm,
                 src_v, dst_v, att_v, lb0, lb1, rb0, rb1, a0, a1,
                 gs0, gs1, ws0, ws1):
    base = _wid() * EPW
    pltpu.sync_copy(src_hbm.at[pl.ds(base, EPW)], src_v)
    pltpu.sync_copy(dst_hbm.at[pl.ds(base, EPW)], dst_v)
    pltpu.sync_copy(att_hbm, att_v)
    zeros16 = jnp.zeros((16,), jnp.float32)
    lane = _lane_iota()
    rots = [(lane + s) & 15 for s in (1, 2, 4, 8)]

    def fire(k, lb, rb, gs):
        pltpu.async_copy(xl_hbm.at[src_v.at[pl.ds(k * GB, GB)]], lb, gs)
        pltpu.async_copy(xr_hbm.at[dst_v.at[pl.ds(k * GB, GB)]], rb, gs)

    def waitg(lb, rb, gs):
        pltpu.make_async_copy(xl_hbm.at[pl.ds(0, GB)], lb, gs).wait()
        pltpu.make_async_copy(xr_hbm.at[pl.ds(0, GB)], rb, gs).wait()

    def compute(k, lb, rb, ab, ws):
        @pl.when(k >= 2)
        def _():
            pltpu.make_async_copy(ab, alpha_hbm.at[pl.ds(base * 16, GB * 16)], ws).wait()

        def edge_body(e, _):
            row = zeros16
            for h in range(H1):
                acc = zeros16
                for v in range(CP // 16):
                    off = h * CP + v * 16
                    t = lb[e, pl.ds(off, 16)] + rb[e, pl.ds(off, 16)]
                    t = jnp.maximum(t, t * 0.2)
                    acc = acc + t * att_v[h, pl.ds(v * 16, 16)]
                for r in rots:
                    acc = acc + _lane_perm(acc, r)
                row = jnp.where(lane == h, acc, row)
            eo = pl.multiple_of(e * 16, 16)
            ab[pl.ds(eo, 16)] = row
            return 0

        lax.fori_loop(0, GB, edge_body, 0)
        pltpu.async_copy(ab, alpha_hbm.at[pl.ds((base + k * GB) * 16, GB * 16)], ws)

    fire(0, lb0, rb0, gs0)
    fire(1, lb1, rb1, gs1)

    def loop_body(bi, _):
        k0 = 2 * bi
        waitg(lb0, rb0, gs0)
        compute(k0, lb0, rb0, a0, ws0)

        @pl.when(k0 + 2 < NB)
        def _():
            fire(k0 + 2, lb0, rb0, gs0)

        waitg(lb1, rb1, gs1)
        compute(k0 + 1, lb1, rb1, a1, ws1)

        @pl.when(k0 + 3 < NB)
        def _():
            fire(k0 + 3, lb1, rb1, gs1)

        return 0

    lax.fori_loop(0, NB // 2, loop_body, 0)
    waitg(lb0, rb0, gs0)
    compute(NB - 1, lb0, rb0, a0, ws0)
    pltpu.make_async_copy(a0, alpha_hbm.at[pl.ds(base * 16, GB * 16)], ws0).wait()
    pltpu.make_async_copy(a1, alpha_hbm.at[pl.ds(base * 16, GB * 16)], ws1).wait()


def _pass_a(xlp, xrp, src, dst, attp):
    kfn = pl.kernel(
        _pass_a_body,
        out_type=jax.ShapeDtypeStruct((E * 16,), jnp.float32),
        mesh=plsc.VectorSubcoreMesh(**_MESH),
        scratch_types=[
            pltpu.VMEM((EPW,), jnp.int32),
            pltpu.VMEM((EPW,), jnp.int32),
            pltpu.VMEM((H1, CP), jnp.float32),
            pltpu.VMEM((GB, D), jnp.float32),
            pltpu.VMEM((GB, D), jnp.float32),
            pltpu.VMEM((GB, D), jnp.float32),
            pltpu.VMEM((GB, D), jnp.float32),
            pltpu.VMEM((GB * 16,), jnp.float32),
            pltpu.VMEM((GB * 16,), jnp.float32),
            pltpu.SemaphoreType.DMA,
            pltpu.SemaphoreType.DMA,
            pltpu.SemaphoreType.DMA,
            pltpu.SemaphoreType.DMA,
        ],
    )
    return kfn(xlp, xrp, src, dst, attp)


def _pass_b_body(dst_hbm, alpha_hbm, mtab_hbm, stab_hbm,
                 dv0, av0, dv1, av1, mtab, stab, cs0, cs1):
    lo = _wid() * NPW
    neg = jnp.full((16,), -1e30, jnp.float32)
    zeros16 = jnp.zeros((16,), jnp.float32)
    lane = _lane_iota()

    def init_body(i, _):
        o = pl.multiple_of(i * 16, 16)
        mtab[pl.ds(o, 16)] = neg
        stab[pl.ds(o, 16)] = zeros16
        return 0

    lax.fori_loop(0, NPW, init_body, 0)

    def fire(ci, dv, av, cs):
        pltpu.async_copy(dst_hbm.at[pl.ds(ci * CH, CH)], dv, cs)
        pltpu.async_copy(alpha_hbm.at[pl.ds(ci * CH * 16, CH * 16)], av, cs)

    def waitc(dv, av, cs):
        pltpu.make_async_copy(dst_hbm.at[pl.ds(0, CH)], dv, cs).wait()
        pltpu.make_async_copy(alpha_hbm.at[pl.ds(0, CH * 16)], av, cs).wait()

    def process(dv, av):
        def group(gi, _):
            gbase = pl.multiple_of(gi * 16, 16)
            d = dv[pl.ds(gbase, 16)]
            dl = d - lo
            mask = (dl >= 0) & (dl < NPW)
            tot = _lane_count(mask)
            lv0 = jnp.where(mask, lane, 16)

            def wbody(k, lv):
                i0v = _lane_min(lv) & 15
                i0 = _to_scalar(i0v)
                dli = _dyn_lane(dl, i0)
                to = pl.multiple_of(dli * 16, 16)
                ao = pl.multiple_of((gbase + i0) * 16, 16)
                arow = av[pl.ds(ao, 16)]
                m0 = mtab[pl.ds(to, 16)]
                mn = jnp.maximum(m0, arow)
                em = jnp.exp(m0 - mn)
                stab[pl.ds(to, 16)] = stab[pl.ds(to, 16)] * em + jnp.exp(arow - mn)
                mtab[pl.ds(to, 16)] = mn
                return jnp.where(lane == i0v, 16, lv)

            lax.fori_loop(0, _to_scalar(tot), wbody, lv0)
            return 0

        lax.fori_loop(0, CH // 16, group, 0)

    fire(0, dv0, av0, cs0)
    fire(1, dv1, av1, cs1)

    def chunk_loop(hi, _):
        c0 = 2 * hi
        waitc(dv0, av0, cs0)
        process(dv0, av0)

        @pl.when(c0 + 2 < NCH)
        def _():
            fire(c0 + 2, dv0, av0, cs0)

        waitc(dv1, av1, cs1)
        process(dv1, av1)

        @pl.when(c0 + 3 < NCH)
        def _():
            fire(c0 + 3, dv1, av1, cs1)

        return 0

    lax.fori_loop(0, NCH // 2, chunk_loop, 0)
    pltpu.sync_copy(mtab, mtab_hbm.at[pl.ds(lo * 16, NPW * 16)])
    pltpu.sync_copy(stab, stab_hbm.at[pl.ds(lo * 16, NPW * 16)])


def _pass_b(dst, alpha):
    kfn = pl.kernel(
        _pass_b_body,
        out_type=[
            jax.ShapeDtypeStruct((NPT * 16,), jnp.float32),
            jax.ShapeDtypeStruct((NPT * 16,), jnp.float32),
        ],
        mesh=plsc.VectorSubcoreMesh(**_MESH),
        scratch_types=[
            pltpu.VMEM((CH,), jnp.int32),
            pltpu.VMEM((CH * 16,), jnp.float32),
            pltpu.VMEM((CH,), jnp.int32),
            pltpu.VMEM((CH * 16,), jnp.float32),
            pltpu.VMEM((NPW * 16,), jnp.float32),
            pltpu.VMEM((NPW * 16,), jnp.float32),
            pltpu.SemaphoreType.DMA,
            pltpu.SemaphoreType.DMA,
        ],
    )
    return kfn(dst, alpha)


def _pass_c_body(xl_hbm, src_hbm, dst_hbm, alpha_hbm, mtab_hbm, stab_hbm,
                 bias_hbm, hout_hbm,
                 sv0, dv0, sv1, dv1, mtv, stv, acc, biasv,
                 idxs, idxw, xst, astw, pdref, psref, pjref, pcref,
                 cs0, cs1, gs0):
    wid = _wid()
    pltpu.sync_copy(bias_hbm, biasv)
    zeros16 = jnp.zeros((16,), jnp.float32)
    zi = jnp.zeros((16,), jnp.int32)
    lane = _lane_iota()

    def flush(cnt, pd, ps, pj):
        idxs[...] = ps

        def widx(e, _):
            o = pl.multiple_of(e * 16, 16)
            pjv = _lane_perm(pj, jnp.zeros((16,), jnp.int32) + e)
            idxw[pl.ds(o, 16)] = pjv * 16 + lane
            return 0

        lax.fori_loop(0, FL, widx, 0)
        cx = pltpu.async_copy(xl_hbm.at[idxs], xst, gs0)
        ca = pltpu.async_copy(alpha_hbm.at[idxw], astw, gs0)
        cx.wait()
        ca.wait()

        def pe(e, _):
            dl_e = _dyn_lane(pd, e)
            to = pl.multiple_of(dl_e * 16, 16)
            ao = pl.multiple_of(e * 16, 16)
            arow = astw[pl.ds(ao, 16)]
            c = jnp.exp(arow - mtv[pl.ds(to, 16)]) / (stv[pl.ds(to, 16)] + 1e-16)
            for h in range(H1):
                chv = _lane_perm(c, jnp.full((16,), h, jnp.int32))
                for v in range(CP // 16):
                    off = h * CP + v * 16
                    acc[dl_e, pl.ds(off, 16)] = (
                        acc[dl_e, pl.ds(off, 16)] + xst[e, pl.ds(off, 16)] * chv)
            return 0

        lax.fori_loop(0, cnt, pe, 0)

    def range_pass(rp, _):
        rng = rp * W + wid

        @pl.when(rng < RC)
        def _():
            lo = rng * NPC
            pltpu.sync_copy(mtab_hbm.at[pl.ds(lo * 16, NPC * 16)], mtv)
            pltpu.sync_copy(stab_hbm.at[pl.ds(lo * 16, NPC * 16)], stv)

            def z(i, _):
                for v in range(D // 16):
                    acc[i, pl.ds(v * 16, 16)] = zeros16
                return 0

            lax.fori_loop(0, NPC, z, 0)

            def do_chunk(c0, sv, dv):
                def group(gi, _):
                    gbase = pl.multiple_of(gi * 16, 16)
                    d = dv[pl.ds(gbase, 16)]
                    sg = sv[pl.ds(gbase, 16)]
                    dl = d - lo
                    mask = (dl >= 0) & (dl < NPC)
                    tot = _lane_count(mask)
                    tot_s = _to_scalar(tot)

                    @pl.when(tot_s > 0)
                    def _():
                        pd = pdref[pl.ds(0, 16)]
                        ps = psref[pl.ds(0, 16)]
                        pj = pjref[pl.ds(0, 16)]
                        pcv = pcref[pl.ds(0, 16)]
                        lv0 = jnp.where(mask, lane, 16)

                        def compact(k, st4):
                            hd, hs, hj, lv = st4
                            i0v = _lane_min(lv) & 15
                            hd = jnp.where(lane == k, _lane_perm(dl, i0v), hd)
                            hs = jnp.where(lane == k, _lane_perm(sg, i0v), hs)
                            hj = jnp.where(lane == k, c0 * CH + gbase + i0v, hj)
                            lv = jnp.where(lane == i0v, 16, lv)
                            return (hd, hs, hj, lv)

                        hd, hs, hj, _lv = lax.fori_loop(
                            0, tot_s, compact, (zi, zi, zi, lv0))
                        shd = _lane_perm(hd, (lane - pcv) & 15)
                        shs = _lane_perm(hs, (lane - pcv) & 15)
                        shj = _lane_perm(hj, (lane - pcv) & 15)
                        pd_n = jnp.where(lane >= pcv, shd, pd)
                        ps_n = jnp.where(lane >= pcv, shs, ps)
                        pj_n = jnp.where(lane >= pcv, shj, pj)
                        mv = pcv + tot

                        @pl.when(_to_scalar(mv) >= FL)
                        def _():
                            flush(FL, pd_n, ps_n, pj_n)

                        pd_a = _lane_perm(hd, (lane + FL - pcv) & 15)
                        ps_a = _lane_perm(hs, (lane + FL - pcv) & 15)
                        pj_a = _lane_perm(hj, (lane + FL - pcv) & 15)
                        ovf = mv >= FL
                        pdref[...] = jnp.where(ovf, pd_a, pd_n)
                        psref[...] = jnp.where(ovf, ps_a, ps_n)
                        pjref[...] = jnp.where(ovf, pj_a, pj_n)
                        pcref[...] = jnp.where(ovf, mv - FL, mv)

                    return 0

                lax.fori_loop(0, CH // 16, group, 0)

            def fire(ci, sv, dv, cs):
                pltpu.async_copy(src_hbm.at[pl.ds(ci * CH, CH)], sv, cs)
                pltpu.async_copy(dst_hbm.at[pl.ds(ci * CH, CH)], dv, cs)

            def waitc(sv, dv, cs):
                pltpu.make_async_copy(src_hbm.at[pl.ds(0, CH)], sv, cs).wait()
                pltpu.make_async_copy(dst_hbm.at[pl.ds(0, CH)], dv, cs).wait()

            pdref[...] = zi
            psref[...] = zi
            pjref[...] = zi
            pcref[...] = zi
            fire(0, sv0, dv0, cs0)
            fire(1, sv1, dv1, cs1)

            def chunk_loop(hi, _):
                c0 = 2 * hi
                waitc(sv0, dv0, cs0)
                do_chunk(c0, sv0, dv0)

                @pl.when(c0 + 2 < NCH)
                def _():
                    fire(c0 + 2, sv0, dv0, cs0)

                waitc(sv1, dv1, cs1)
                do_chunk(c0 + 1, sv1, dv1)

                @pl.when(c0 + 3 < NCH)
                def _():
                    fire(c0 + 3, sv1, dv1, cs1)

                return 0

            lax.fori_loop(0, NCH // 2, chunk_loop, 0)
            pc = _to_scalar(pcref[pl.ds(0, 16)])

            @pl.when(pc > 0)
            def _():
                flush(pc, pdref[pl.ds(0, 16)], psref[pl.ds(0, 16)],
                      pjref[pl.ds(0, 16)])

            def ep(i, _):
                for v in range(D // 16):
                    o = v * 16
                    acc[i, pl.ds(o, 16)] = jnp.maximum(
                        acc[i, pl.ds(o, 16)] + biasv[pl.ds(o, 16)], 0.0)
                return 0

            lax.fori_loop(0, NPC, ep, 0)
            pltpu.sync_copy(acc, hout_hbm.at[pl.ds(lo, NPC)])

        return 0

    lax.fori_loop(0, (RC + W - 1) // W, range_pass, 0)


def _pass_c(xlp, src, dst, alpha, mtab, stab, biasp):
    kfn = pl.kernel(
        _pass_c_body,
        out_type=jax.ShapeDtypeStruct((N, D), jnp.float32),
        mesh=plsc.VectorSubcoreMesh(**_MESH),
        scratch_types=[
            pltpu.VMEM((CH,), jnp.int32),
            pltpu.VMEM((CH,), jnp.int32),
            pltpu.VMEM((CH,), jnp.int32),
            pltpu.VMEM((CH,), jnp.int32),
            pltpu.VMEM((NPC * 16,), jnp.float32),
            pltpu.VMEM((NPC * 16,), jnp.float32),
            pltpu.VMEM((NPC, D), jnp.float32),
            pltpu.VMEM((D,), jnp.float32),
            pltpu.VMEM((FL,), jnp.int32),
            pltpu.VMEM((FL * 16,), jnp.int32),
            pltpu.VMEM((FL, D), jnp.float32),
            pltpu.VMEM((FL * 16,), jnp.float32),
            pltpu.VMEM((16,), jnp.int32),
            pltpu.VMEM((16,), jnp.int32),
            pltpu.VMEM((16,), jnp.int32),
            pltpu.VMEM((16,), jnp.int32),
            pltpu.SemaphoreType.DMA,
            pltpu.SemaphoreType.DMA,
            pltpu.SemaphoreType.DMA,
        ],
    )
    return kfn(xlp, src, dst, alpha, mtab, stab, biasp)


def _pass_d_body(src_hbm, dst_hbm, xl2_hbm, xr2_hbm, sc2_hbm, out_hbm,
                 sv0, dv0, sv1, dv1, xl2v, xr2v, mt, st, vt, sc2v, cs0, cs1):
    lo = _wid() * NPW
    neg = jnp.full((16,), -1e30, jnp.float32)
    zeros16 = jnp.zeros((16,), jnp.float32)
    lane = _lane_iota()
    pltpu.sync_copy(xl2_hbm, xl2v)
    pltpu.sync_copy(xr2_hbm, xr2v)
    pltpu.sync_copy(sc2_hbm, sc2v)
    scv = sc2v[pl.ds(0, 16)]
    att2s = scv[0]
    bias2s = scv[1]

    def init_body(i, _):
        o = pl.multiple_of(i * 16, 16)
        mt[pl.ds(o, 16)] = neg
        st[pl.ds(o, 16)] = zeros16
        vt[pl.ds(o, 16)] = zeros16
        return 0

    lax.fori_loop(0, NPW, init_body, 0)

    def fire(ci, sv, dv, cs):
        pltpu.async_copy(src_hbm.at[pl.ds(ci * CH, CH)], sv, cs)
        pltpu.async_copy(dst_hbm.at[pl.ds(ci * CH, CH)], dv, cs)

    def waitc(sv, dv, cs):
        pltpu.make_async_copy(src_hbm.at[pl.ds(0, CH)], sv, cs).wait()
        pltpu.make_async_copy(dst_hbm.at[pl.ds(0, CH)], dv, cs).wait()

    def tab_read(tab, i):
        ib = pl.multiple_of((i >> 4) << 4, 16)
        return _dyn_lane(tab[pl.ds(ib, 16)], i & 15)

    def process(sv, dv):
        def group(gi, _):
            gbase = pl.multiple_of(gi * 16, 16)
            d = dv[pl.ds(gbase, 16)]
            sg = sv[pl.ds(gbase, 16)]
            dl = d - lo
            mask = (dl >= 0) & (dl < NPW)
            tot = _lane_count(mask)
            lv0 = jnp.where(mask, lane, 16)

            def wbody(k, lv):
                i0 = _to_scalar(_lane_min(lv))
                dli = _dyn_lane(dl, i0)
                si = _dyn_lane(sg, i0)
                di = _dyn_lane(d, i0)
                xls = tab_read(xl2v, si)
                t = xls + tab_read(xr2v, di)
                a2 = att2s * jnp.maximum(t, 0.2 * t)
                to = pl.multiple_of(dli * 16, 16)
                m0 = mt[pl.ds(to, 16)]
                mn = jnp.maximum(m0, a2)
                em = jnp.exp(m0 - mn)
                ea = jnp.exp(a2 - mn)
                st[pl.ds(to, 16)] = st[pl.ds(to, 16)] * em + ea
                vt[pl.ds(to, 16)] = vt[pl.ds(to, 16)] * em + ea * xls
                mt[pl.ds(to, 16)] = mn
                return jnp.where(lane == i0, 16, lv)

            lax.fori_loop(0, _to_scalar(tot), wbody, lv0)
            return 0

        lax.fori_loop(0, CH // 16, group, 0)

    fire(0, sv0, dv0, cs0)
    fire(1, sv1, dv1, cs1)

    def chunk_loop(hi, _):
        c0 = 2 * hi
        waitc(sv0, dv0, cs0)
        process(sv0, dv0)

        @pl.when(c0 + 2 < NCH)
        def _():
            fire(c0 + 2, sv0, dv0, cs0)

        waitc(sv1, dv1, cs1)
        process(sv1, dv1)

        @pl.when(c0 + 3 < NCH)
        def _():
            fire(c0 + 3, sv1, dv1, cs1)

        return 0

    lax.fori_loop(0, NCH // 2, chunk_loop, 0)

    def ep(i, _):
        o = pl.multiple_of(i * 16, 16)
        vt[pl.ds(o, 16)] = vt[pl.ds(o, 16)] / (st[pl.ds(o, 16)] + 1e-16) + bias2s
        return 0

    lax.fori_loop(0, NPW, ep, 0)
    pltpu.sync_copy(vt, out_hbm.at[pl.ds(lo * 16, NPW * 16)])


def _pass_d(src, dst, xl2, xr2, sc2):
    kfn = pl.kernel(
        _pass_d_body,
        out_type=jax.ShapeDtypeStruct((NPT * 16,), jnp.float32),
        mesh=plsc.VectorSubcoreMesh(**_MESH),
        scratch_types=[
            pltpu.VMEM((CH,), jnp.int32),
            pltpu.VMEM((CH,), jnp.int32),
            pltpu.VMEM((CH,), jnp.int32),
            pltpu.VMEM((CH,), jnp.int32),
            pltpu.VMEM((NPT,), jnp.float32),
            pltpu.VMEM((NPT,), jnp.float32),
            pltpu.VMEM((NPW * 16,), jnp.float32),
            pltpu.VMEM((NPW * 16,), jnp.float32),
            pltpu.VMEM((NPW * 16,), jnp.float32),
            pltpu.VMEM((16,), jnp.float32),
            pltpu.SemaphoreType.DMA,
            pltpu.SemaphoreType.DMA,
        ],
    )
    return kfn(src, dst, xl2, xr2, sc2)


def _pad_heads(w, heads, ch):
    lead = w.shape[:-1]
    w = w.reshape(lead + (heads, ch))
    w = jnp.pad(w, [(0, 0)] * len(lead) + [(0, 0), (0, CP - ch)])
    return w.reshape(lead + (heads * CP,))


def kernel(x, edge_index, Wl1, bl1, Wr1, br1, att1, bias1, Wl2, bl2, Wr2, br2, att2, bias2):
    src = edge_index[0]
    dst = edge_index[1]

    Wl1p = _pad_heads(Wl1, H1, C1)
    Wr1p = _pad_heads(Wr1, H1, C1)
    bl1p = _pad_heads(bl1, H1, C1)
    br1p = _pad_heads(br1, H1, C1)
    attp = _pad_heads(att1.reshape(1, H1 * C1), H1, C1).reshape(H1, CP)
    bias1p = _pad_heads(bias1, H1, C1)

    xlp, xrp = _proj(x, Wl1p, bl1p, Wr1p, br1p)
    alpha = _pass_a(xlp, xrp, src, dst, attp)
    mtab, stab = _pass_b(dst, alpha)
    hp = _pass_c(xlp, src, dst, alpha, mtab, stab, bias1p)

    Wl2p = _pad_heads(Wl2.T, H1, C1).T
    Wr2p = _pad_heads(Wr2.T, H1, C1).T
    xl2, xr2 = _proj(hp, Wl2p, bl2, Wr2p, br2)
    xl2p = jnp.pad(xl2[:, 0], (0, NPT - N))
    xr2p = jnp.pad(xr2[:, 0], (0, NPT - N))
    sc2 = jnp.concatenate([att2.reshape(1), bias2.reshape(1),
                           jnp.zeros((14,), jnp.float32)])
    out2 = _pass_d(src, dst, xl2p, xr2p, sc2)
    return out2.reshape(NPT, 16)[:N, 0:1]


# speedup vs baseline: 4.6469x; 1.1522x over previous
import jax
import jax.numpy as jnp
from jax import lax
from jax.experimental import pallas as pl
from jax.experimental.pallas import tpu as pltpu
from jax.experimental.pallas import tpu_sc as plsc

N = 10000
E = 160000
H1, C1 = 8, 120
CP = 128
D = H1 * CP
W = 32
EPW = E // W
GB = 8
NB = EPW // GB
NPW = 320
NPT = W * NPW
CH = 2000
NCH = E // CH
RC = 125
NPC = N // RC
FL = 16

_MESH = dict(core_axis_name="c", subcore_axis_name="s")
_GDN = lax.GatherDimensionNumbers(
    offset_dims=(), collapsed_slice_dims=(0,), start_index_map=(0,))


def _lane_perm(x, idx):
    return lax.gather(x, idx[:, None], _GDN, (1,),
                      mode=lax.GatherScatterMode.PROMISE_IN_BOUNDS)


def _lane_iota():
    return lax.broadcasted_iota(jnp.int32, (16,), 0)


def _to_scalar(v):
    return jnp.where(_lane_iota() == 0, v, jnp.zeros_like(v))[0]


def _dyn_lane(v, i):
    return _to_scalar(_lane_perm(v, jnp.zeros((16,), jnp.int32) + i))


def _lane_count(mask):
    lane = _lane_iota()
    x = jnp.where(mask, jnp.ones((16,), jnp.int32), jnp.zeros((16,), jnp.int32))
    for sft in (1, 2, 4, 8):
        x = x + _lane_perm(x, (lane + sft) & 15)
    return x


def _lane_min(x):
    lane = _lane_iota()
    for sft in (1, 2, 4, 8):
        x = jnp.minimum(x, _lane_perm(x, (lane + sft) & 15))
    return x


def _wid():
    return lax.axis_index("s") * 2 + lax.axis_index("c")


def _proj_kernel(x_ref, wl_ref, wr_ref, bl_ref, br_ref, xl_ref, xr_ref):
    x = x_ref[...]
    xl_ref[...] = jnp.dot(x, wl_ref[...], preferred_element_type=jnp.float32) + bl_ref[...]
    xr_ref[...] = jnp.dot(x, wr_ref[...], preferred_element_type=jnp.float32) + br_ref[...]


def _proj(x, Wl, bl, Wr, br):
    n, f = x.shape
    k = Wl.shape[1]
    blk = 1000
    return pl.pallas_call(
        _proj_kernel,
        grid=(n // blk,),
        in_specs=[
            pl.BlockSpec((blk, f), lambda i: (i, 0)),
            pl.BlockSpec((f, k), lambda i: (0, 0)),
            pl.BlockSpec((f, k), lambda i: (0, 0)),
            pl.BlockSpec((k,), lambda i: (0,)),
            pl.BlockSpec((k,), lambda i: (0,)),
        ],
        out_specs=[
            pl.BlockSpec((blk, k), lambda i: (i, 0)),
            pl.BlockSpec((blk, k), lambda i: (i, 0)),
        ],
        out_shape=[
            jax.ShapeDtypeStruct((n, k), jnp.float32),
            jax.ShapeDtypeStruct((n, k), jnp.float32),
        ],
    )(x, Wl, Wr, bl, br)


def _pass_a_body(xl_hbm, xr_hbm, src_hbm, dst_hbm, att_hbm, alpha_hbm,
                 src_v, dst_v, att_v, lb0, lb1, rb0, rb1, a0, a1,
                 gs0, gs1, ws0, ws1):
    base = _wid() * EPW
    pltpu.sync_copy(src_hbm.at[pl.ds(base, EPW)], src_v)
    pltpu.sync_copy(dst_hbm.at[pl.ds(base, EPW)], dst_v)
    pltpu.sync_copy(att_hbm, att_v)
    zeros16 = jnp.zeros((16,), jnp.float32)
    lane = _lane_iota()
    rots = [(lane + s) & 15 for s in (1, 2, 4, 8)]

    def fire(k, lb, rb, gs):
        pltpu.async_copy(xl_hbm.at[src_v.at[pl.ds(k * GB, GB)]], lb, gs)
        pltpu.async_copy(xr_hbm.at[dst_v.at[pl.ds(k * GB, GB)]], rb, gs)

    def waitg(lb, rb, gs):
        pltpu.make_async_copy(xl_hbm.at[pl.ds(0, GB)], lb, gs).wait()
        pltpu.make_async_copy(xr_hbm.at[pl.ds(0, GB)], rb, gs).wait()

    def compute(k, lb, rb, ab, ws):
        @pl.when(k >= 2)
        def _():
            pltpu.make_async_copy(ab, alpha_hbm.at[pl.ds(base * 16, GB * 16)], ws).wait()

        def edge_body(e, _):
            row = zeros16
            for h in range(H1):
                acc = zeros16
                for v in range(CP // 16):
                    off = h * CP + v * 16
                    t = lb[e, pl.ds(off, 16)] + rb[e, pl.ds(off, 16)]
                    t = jnp.maximum(t, t * 0.2)
                    acc = acc + t * att_v[h, pl.ds(v * 16, 16)]
                for r in rots:
                    acc = acc + _lane_perm(acc, r)
                row = jnp.where(lane == h, acc, row)
            eo = pl.multiple_of(e * 16, 16)
            ab[pl.ds(eo, 16)] = row
            return 0

        lax.fori_loop(0, GB, edge_body, 0)
        pltpu.async_copy(ab, alpha_hbm.at[pl.ds((base + k * GB) * 16, GB * 16)], ws)

    fire(0, lb0, rb0, gs0)
    fire(1, lb1, rb1, gs1)

    def loop_body(bi, _):
        k0 = 2 * bi
        waitg(lb0, rb0, gs0)
        compute(k0, lb0, rb0, a0, ws0)

        @pl.when(k0 + 2 < NB)
        def _():
            fire(k0 + 2, lb0, rb0, gs0)

        waitg(lb1, rb1, gs1)
        compute(k0 + 1, lb1, rb1, a1, ws1)

        @pl.when(k0 + 3 < NB)
        def _():
            fire(k0 + 3, lb1, rb1, gs1)

        return 0

    lax.fori_loop(0, NB // 2, loop_body, 0)
    waitg(lb0, rb0, gs0)
    compute(NB - 1, lb0, rb0, a0, ws0)
    pltpu.make_async_copy(a0, alpha_hbm.at[pl.ds(base * 16, GB * 16)], ws0).wait()
    pltpu.make_async_copy(a1, alpha_hbm.at[pl.ds(base * 16, GB * 16)], ws1).wait()


def _pass_a(xlp, xrp, src, dst, attp):
    kfn = pl.kernel(
        _pass_a_body,
        out_type=jax.ShapeDtypeStruct((E * 16,), jnp.float32),
        mesh=plsc.VectorSubcoreMesh(**_MESH),
        scratch_types=[
            pltpu.VMEM((EPW,), jnp.int32),
            pltpu.VMEM((EPW,), jnp.int32),
            pltpu.VMEM((H1, CP), jnp.float32),
            pltpu.VMEM((GB, D), jnp.float32),
            pltpu.VMEM((GB, D), jnp.float32),
            pltpu.VMEM((GB, D), jnp.float32),
            pltpu.VMEM((GB, D), jnp.float32),
            pltpu.VMEM((GB * 16,), jnp.float32),
            pltpu.VMEM((GB * 16,), jnp.float32),
            pltpu.SemaphoreType.DMA,
            pltpu.SemaphoreType.DMA,
            pltpu.SemaphoreType.DMA,
            pltpu.SemaphoreType.DMA,
        ],
    )
    return kfn(xlp, xrp, src, dst, attp)


def _pass_b_body(dst_hbm, alpha_hbm, mtab_hbm, stab_hbm,
                 dv0, av0, dv1, av1, mtab, stab, cs0, cs1):
    lo = _wid() * NPW
    neg = jnp.full((16,), -1e30, jnp.float32)
    zeros16 = jnp.zeros((16,), jnp.float32)
    lane = _lane_iota()

    def init_body(i, _):
        o = pl.multiple_of(i * 16, 16)
        mtab[pl.ds(o, 16)] = neg
        stab[pl.ds(o, 16)] = zeros16
        return 0

    lax.fori_loop(0, NPW, init_body, 0)

    def fire(ci, dv, av, cs):
        pltpu.async_copy(dst_hbm.at[pl.ds(ci * CH, CH)], dv, cs)
        pltpu.async_copy(alpha_hbm.at[pl.ds(ci * CH * 16, CH * 16)], av, cs)

    def waitc(dv, av, cs):
        pltpu.make_async_copy(dst_hbm.at[pl.ds(0, CH)], dv, cs).wait()
        pltpu.make_async_copy(alpha_hbm.at[pl.ds(0, CH * 16)], av, cs).wait()

    def process(dv, av):
        def group(gi, _):
            gbase = pl.multiple_of(gi * 16, 16)
            d = dv[pl.ds(gbase, 16)]
            dl = d - lo
            mask = (dl >= 0) & (dl < NPW)
            tot = _lane_count(mask)
            lv0 = jnp.where(mask, lane, 16)

            def wbody(k, lv):
                i0v = _lane_min(lv) & 15
                i0 = _to_scalar(i0v)
                dli = _dyn_lane(dl, i0)
                to = pl.multiple_of(dli * 16, 16)
                ao = pl.multiple_of((gbase + i0) * 16, 16)
                arow = av[pl.ds(ao, 16)]
                m0 = mtab[pl.ds(to, 16)]
                mn = jnp.maximum(m0, arow)
                em = jnp.exp(m0 - mn)
                stab[pl.ds(to, 16)] = stab[pl.ds(to, 16)] * em + jnp.exp(arow - mn)
                mtab[pl.ds(to, 16)] = mn
                return jnp.where(lane == i0v, 16, lv)

            lax.fori_loop(0, _to_scalar(tot), wbody, lv0)
            return 0

        lax.fori_loop(0, CH // 16, group, 0)

    fire(0, dv0, av0, cs0)
    fire(1, dv1, av1, cs1)

    def chunk_loop(hi, _):
        c0 = 2 * hi
        waitc(dv0, av0, cs0)
        process(dv0, av0)

        @pl.when(c0 + 2 < NCH)
        def _():
            fire(c0 + 2, dv0, av0, cs0)

        waitc(dv1, av1, cs1)
        process(dv1, av1)

        @pl.when(c0 + 3 < NCH)
        def _():
            fire(c0 + 3, dv1, av1, cs1)

        return 0

    lax.fori_loop(0, NCH // 2, chunk_loop, 0)
    pltpu.sync_copy(mtab, mtab_hbm.at[pl.ds(lo * 16, NPW * 16)])
    pltpu.sync_copy(stab, stab_hbm.at[pl.ds(lo * 16, NPW * 16)])


def _pass_b(dst, alpha):
    kfn = pl.kernel(
        _pass_b_body,
        out_type=[
            jax.ShapeDtypeStruct((NPT * 16,), jnp.float32),
            jax.ShapeDtypeStruct((NPT * 16,), jnp.float32),
        ],
        mesh=plsc.VectorSubcoreMesh(**_MESH),
        scratch_types=[
            pltpu.VMEM((CH,), jnp.int32),
            pltpu.VMEM((CH * 16,), jnp.float32),
            pltpu.VMEM((CH,), jnp.int32),
            pltpu.VMEM((CH * 16,), jnp.float32),
            pltpu.VMEM((NPW * 16,), jnp.float32),
            pltpu.VMEM((NPW * 16,), jnp.float32),
            pltpu.SemaphoreType.DMA,
            pltpu.SemaphoreType.DMA,
        ],
    )
    return kfn(dst, alpha)


def _pass_c_body(xl_hbm, src_hbm, dst_hbm, alpha_hbm, mtab_hbm, stab_hbm,
                 bias_hbm, hout_hbm,
                 sv0, dv0, sv1, dv1, mtv, stv, acc, biasv,
                 idxs, idxw, xst, astw, pdref, psref, pjref, pcref,
                 fdref, fcref, cs0, cs1, gs0):
    wid = _wid()
    pltpu.sync_copy(bias_hbm, biasv)
    zeros16 = jnp.zeros((16,), jnp.float32)
    zi = jnp.zeros((16,), jnp.int32)
    lane = _lane_iota()

    def flush_fire(pd, ps, pj):
        idxs[...] = ps

        def widx(e, _):
            o = pl.multiple_of(e * 16, 16)
            pjv = _lane_perm(pj, jnp.zeros((16,), jnp.int32) + e)
            idxw[pl.ds(o, 16)] = pjv * 16 + lane
            return 0

        lax.fori_loop(0, FL, widx, 0)
        pltpu.async_copy(xl_hbm.at[idxs], xst, gs0)
        pltpu.async_copy(alpha_hbm.at[idxw], astw, gs0)

    def flush_drain(cnt, pd):
        pltpu.make_async_copy(xl_hbm.at[pl.ds(0, FL)], xst, gs0).wait()
        pltpu.make_async_copy(alpha_hbm.at[idxw], astw, gs0).wait()

        def pe(e, _):
            dl_e = _dyn_lane(pd, e)
            to = pl.multiple_of(dl_e * 16, 16)
            ao = pl.multiple_of(e * 16, 16)
            arow = astw[pl.ds(ao, 16)]
            c = jnp.exp(arow - mtv[pl.ds(to, 16)]) / (stv[pl.ds(to, 16)] + 1e-16)
            for h in range(H1):
                chv = _lane_perm(c, jnp.full((16,), h, jnp.int32))
                for v in range(CP // 16):
                    off = h * CP + v * 16
                    plsc.addupdate(acc.at[dl_e, pl.ds(off, 16)],
                                   xst[e, pl.ds(off, 16)] * chv)
            return 0

        lax.fori_loop(0, cnt, pe, 0)

    def range_pass(rp, _):
        rng = rp * W + wid

        @pl.when(rng < RC)
        def _():
            lo = rng * NPC
            pltpu.sync_copy(mtab_hbm.at[pl.ds(lo * 16, NPC * 16)], mtv)
            pltpu.sync_copy(stab_hbm.at[pl.ds(lo * 16, NPC * 16)], stv)

            def z(i, _):
                for v in range(D // 16):
                    acc[i, pl.ds(v * 16, 16)] = zeros16
                return 0

            lax.fori_loop(0, NPC, z, 0)

            def do_chunk(c0, sv, dv):
                def group(gi, _):
                    gbase = pl.multiple_of(gi * 16, 16)
                    d = dv[pl.ds(gbase, 16)]
                    sg = sv[pl.ds(gbase, 16)]
                    dl = d - lo
                    mask = (dl >= 0) & (dl < NPC)
                    tot = _lane_count(mask)
                    tot_s = _to_scalar(tot)

                    @pl.when(tot_s > 0)
                    def _():
                        pd = pdref[pl.ds(0, 16)]
                        ps = psref[pl.ds(0, 16)]
                        pj = pjref[pl.ds(0, 16)]
                        pcv = pcref[pl.ds(0, 16)]
                        lv0 = jnp.where(mask, lane, 16)

                        def compact(k, st4):
                            hd, hs, hj, lv = st4
                            i0v = _lane_min(lv) & 15
                            hd = jnp.where(lane == k, _lane_perm(dl, i0v), hd)
                            hs = jnp.where(lane == k, _lane_perm(sg, i0v), hs)
                            hj = jnp.where(lane == k, c0 * CH + gbase + i0v, hj)
                            lv = jnp.where(lane == i0v, 16, lv)
                            return (hd, hs, hj, lv)

                        hd, hs, hj, _lv = lax.fori_loop(
                            0, tot_s, compact, (zi, zi, zi, lv0))
                        shd = _lane_perm(hd, (lane - pcv) & 15)
                        shs = _lane_perm(hs, (lane - pcv) & 15)
                        shj = _lane_perm(hj, (lane - pcv) & 15)
                        pd_n = jnp.where(lane >= pcv, shd, pd)
                        ps_n = jnp.where(lane >= pcv, shs, ps)
                        pj_n = jnp.where(lane >= pcv, shj, pj)
                        mv = pcv + tot

                        @pl.when(_to_scalar(mv) >= FL)
                        def _():
                            fc = _to_scalar(fcref[pl.ds(0, 16)])

                            @pl.when(fc > 0)
                            def _():
                                flush_drain(FL, fdref[pl.ds(0, 16)])

                            flush_fire(pd_n, ps_n, pj_n)
                            fdref[...] = pd_n
                            fcref[...] = jnp.zeros((16,), jnp.int32) + FL

                        pd_a = _lane_perm(hd, (lane + FL - pcv) & 15)
                        ps_a = _lane_perm(hs, (lane + FL - pcv) & 15)
                        pj_a = _lane_perm(hj, (lane + FL - pcv) & 15)
                        ovf = mv >= FL
                        pdref[...] = jnp.where(ovf, pd_a, pd_n)
                        psref[...] = jnp.where(ovf, ps_a, ps_n)
                        pjref[...] = jnp.where(ovf, pj_a, pj_n)
                        pcref[...] = jnp.where(ovf, mv - FL, mv)

                    return 0

                lax.fori_loop(0, CH // 16, group, 0)

            def fire(ci, sv, dv, cs):
                pltpu.async_copy(src_hbm.at[pl.ds(ci * CH, CH)], sv, cs)
                pltpu.async_copy(dst_hbm.at[pl.ds(ci * CH, CH)], dv, cs)

            def waitc(sv, dv, cs):
                pltpu.make_async_copy(src_hbm.at[pl.ds(0, CH)], sv, cs).wait()
                pltpu.make_async_copy(dst_hbm.at[pl.ds(0, CH)], dv, cs).wait()

            pdref[...] = zi
            psref[...] = zi
            pjref[...] = zi
            pcref[...] = zi
            fcref[...] = zi
            fire(0, sv0, dv0, cs0)
            fire(1, sv1, dv1, cs1)

            def chunk_loop(hi, _):
                c0 = 2 * hi
                waitc(sv0, dv0, cs0)
                do_chunk(c0, sv0, dv0)

                @pl.when(c0 + 2 < NCH)
                def _():
                    fire(c0 + 2, sv0, dv0, cs0)

                waitc(sv1, dv1, cs1)
                do_chunk(c0 + 1, sv1, dv1)

                @pl.when(c0 + 3 < NCH)
                def _():
                    fire(c0 + 3, sv1, dv1, cs1)

                return 0

            lax.fori_loop(0, NCH // 2, chunk_loop, 0)
            fc = _to_scalar(fcref[pl.ds(0, 16)])

            @pl.when(fc > 0)
            def _():
                flush_drain(FL, fdref[pl.ds(0, 16)])

            pc = _to_scalar(pcref[pl.ds(0, 16)])

            @pl.when(pc > 0)
            def _():
                flush_fire(pdref[pl.ds(0, 16)], psref[pl.ds(0, 16)],
                           pjref[pl.ds(0, 16)])
                flush_drain(pc, pdref[pl.ds(0, 16)])

            def ep(i, _):
                for v in range(D // 16):
                    o = v * 16
                    acc[i, pl.ds(o, 16)] = jnp.maximum(
                        acc[i, pl.ds(o, 16)] + biasv[pl.ds(o, 16)], 0.0)
                return 0

            lax.fori_loop(0, NPC, ep, 0)
            pltpu.sync_copy(acc, hout_hbm.at[pl.ds(lo, NPC)])

        return 0

    lax.fori_loop(0, (RC + W - 1) // W, range_pass, 0)


def _pass_c(xlp, src, dst, alpha, mtab, stab, biasp):
    kfn = pl.kernel(
        _pass_c_body,
        out_type=jax.ShapeDtypeStruct((N, D), jnp.float32),
        mesh=plsc.VectorSubcoreMesh(**_MESH),
        scratch_types=[
            pltpu.VMEM((CH,), jnp.int32),
            pltpu.VMEM((CH,), jnp.int32),
            pltpu.VMEM((CH,), jnp.int32),
            pltpu.VMEM((CH,), jnp.int32),
            pltpu.VMEM((NPC * 16,), jnp.float32),
            pltpu.VMEM((NPC * 16,), jnp.float32),
            pltpu.VMEM((NPC, D), jnp.float32),
            pltpu.VMEM((D,), jnp.float32),
            pltpu.VMEM((FL,), jnp.int32),
            pltpu.VMEM((FL * 16,), jnp.int32),
            pltpu.VMEM((FL, D), jnp.float32),
            pltpu.VMEM((FL * 16,), jnp.float32),
            pltpu.VMEM((16,), jnp.int32),
            pltpu.VMEM((16,), jnp.int32),
            pltpu.VMEM((16,), jnp.int32),
            pltpu.VMEM((16,), jnp.int32),
            pltpu.VMEM((16,), jnp.int32),
            pltpu.VMEM((16,), jnp.int32),
            pltpu.SemaphoreType.DMA,
            pltpu.SemaphoreType.DMA,
            pltpu.SemaphoreType.DMA,
        ],
    )
    return kfn(xlp, src, dst, alpha, mtab, stab, biasp)


def _pass_d_body(src_hbm, dst_hbm, xl2_hbm, xr2_hbm, sc2_hbm, out_hbm,
                 sv0, dv0, sv1, dv1, xl2v, xr2v, mt, st, vt, sc2v, cs0, cs1):
    lo = _wid() * NPW
    neg = jnp.full((16,), -1e30, jnp.float32)
    zeros16 = jnp.zeros((16,), jnp.float32)
    lane = _lane_iota()
    pltpu.sync_copy(xl2_hbm, xl2v)
    pltpu.sync_copy(xr2_hbm, xr2v)
    pltpu.sync_copy(sc2_hbm, sc2v)
    scv = sc2v[pl.ds(0, 16)]
    att2s = scv[0]
    bias2s = scv[1]

    def init_body(i, _):
        o = pl.multiple_of(i * 16, 16)
        mt[pl.ds(o, 16)] = neg
        st[pl.ds(o, 16)] = zeros16
        vt[pl.ds(o, 16)] = zeros16
        return 0

    lax.fori_loop(0, NPW, init_body, 0)

    def fire(ci, sv, dv, cs):
        pltpu.async_copy(src_hbm.at[pl.ds(ci * CH, CH)], sv, cs)
        pltpu.async_copy(dst_hbm.at[pl.ds(ci * CH, CH)], dv, cs)

    def waitc(sv, dv, cs):
        pltpu.make_async_copy(src_hbm.at[pl.ds(0, CH)], sv, cs).wait()
        pltpu.make_async_copy(dst_hbm.at[pl.ds(0, CH)], dv, cs).wait()

    def tab_read(tab, i):
        ib = pl.multiple_of((i >> 4) << 4, 16)
        return _dyn_lane(tab[pl.ds(ib, 16)], i & 15)

    def process(sv, dv):
        def group(gi, _):
            gbase = pl.multiple_of(gi * 16, 16)
            d = dv[pl.ds(gbase, 16)]
            sg = sv[pl.ds(gbase, 16)]
            dl = d - lo
            mask = (dl >= 0) & (dl < NPW)
            tot = _lane_count(mask)
            lv0 = jnp.where(mask, lane, 16)

            def wbody(k, lv):
                i0 = _to_scalar(_lane_min(lv))
                dli = _dyn_lane(dl, i0)
                si = _dyn_lane(sg, i0)
                di = _dyn_lane(d, i0)
                xls = tab_read(xl2v, si)
                t = xls + tab_read(xr2v, di)
                a2 = att2s * jnp.maximum(t, 0.2 * t)
                to = pl.multiple_of(dli * 16, 16)
                m0 = mt[pl.ds(to, 16)]
                mn = jnp.maximum(m0, a2)
                em = jnp.exp(m0 - mn)
                ea = jnp.exp(a2 - mn)
                st[pl.ds(to, 16)] = st[pl.ds(to, 16)] * em + ea
                vt[pl.ds(to, 16)] = vt[pl.ds(to, 16)] * em + ea * xls
                mt[pl.ds(to, 16)] = mn
                return jnp.where(lane == i0, 16, lv)

            lax.fori_loop(0, _to_scalar(tot), wbody, lv0)
            return 0

        lax.fori_loop(0, CH // 16, group, 0)

    fire(0, sv0, dv0, cs0)
    fire(1, sv1, dv1, cs1)

    def chunk_loop(hi, _):
        c0 = 2 * hi
        waitc(sv0, dv0, cs0)
        process(sv0, dv0)

        @pl.when(c0 + 2 < NCH)
        def _():
            fire(c0 + 2, sv0, dv0, cs0)

        waitc(sv1, dv1, cs1)
        process(sv1, dv1)

        @pl.when(c0 + 3 < NCH)
        def _():
            fire(c0 + 3, sv1, dv1, cs1)

        return 0

    lax.fori_loop(0, NCH // 2, chunk_loop, 0)

    def ep(i, _):
        o = pl.multiple_of(i * 16, 16)
        vt[pl.ds(o, 16)] = vt[pl.ds(o, 16)] / (st[pl.ds(o, 16)] + 1e-16) + bias2s
        return 0

    lax.fori_loop(0, NPW, ep, 0)
    pltpu.sync_copy(vt, out_hbm.at[pl.ds(lo * 16, NPW * 16)])


def _pass_d(src, dst, xl2, xr2, sc2):
    kfn = pl.kernel(
        _pass_d_body,
        out_type=jax.ShapeDtypeStruct((NPT * 16,), jnp.float32),
        mesh=plsc.VectorSubcoreMesh(**_MESH),
        scratch_types=[
            pltpu.VMEM((CH,), jnp.int32),
            pltpu.VMEM((CH,), jnp.int32),
            pltpu.VMEM((CH,), jnp.int32),
            pltpu.VMEM((CH,), jnp.int32),
            pltpu.VMEM((NPT,), jnp.float32),
            pltpu.VMEM((NPT,), jnp.float32),
            pltpu.VMEM((NPW * 16,), jnp.float32),
            pltpu.VMEM((NPW * 16,), jnp.float32),
            pltpu.VMEM((NPW * 16,), jnp.float32),
            pltpu.VMEM((16,), jnp.float32),
            pltpu.SemaphoreType.DMA,
            pltpu.SemaphoreType.DMA,
        ],
    )
    return kfn(src, dst, xl2, xr2, sc2)


def _pad_heads(w, heads, ch):
    lead = w.shape[:-1]
    w = w.reshape(lead + (heads, ch))
    w = jnp.pad(w, [(0, 0)] * len(lead) + [(0, 0), (0, CP - ch)])
    return w.reshape(lead + (heads * CP,))


def kernel(x, edge_index, Wl1, bl1, Wr1, br1, att1, bias1, Wl2, bl2, Wr2, br2, att2, bias2):
    src = edge_index[0]
    dst = edge_index[1]

    Wl1p = _pad_heads(Wl1, H1, C1)
    Wr1p = _pad_heads(Wr1, H1, C1)
    bl1p = _pad_heads(bl1, H1, C1)
    br1p = _pad_heads(br1, H1, C1)
    attp = _pad_heads(att1.reshape(1, H1 * C1), H1, C1).reshape(H1, CP)
    bias1p = _pad_heads(bias1, H1, C1)

    xlp, xrp = _proj(x, Wl1p, bl1p, Wr1p, br1p)
    alpha = _pass_a(xlp, xrp, src, dst, attp)
    mtab, stab = _pass_b(dst, alpha)
    hp = _pass_c(xlp, src, dst, alpha, mtab, stab, bias1p)

    Wl2p = _pad_heads(Wl2.T, H1, C1).T
    Wr2p = _pad_heads(Wr2.T, H1, C1).T
    xl2, xr2 = _proj(hp, Wl2p, bl2, Wr2p, br2)
    xl2p = jnp.pad(xl2[:, 0], (0, NPT - N))
    xr2p = jnp.pad(xr2[:, 0], (0, NPT - N))
    sc2 = jnp.concatenate([att2.reshape(1), bias2.reshape(1),
                           jnp.zeros((14,), jnp.float32)])
    out2 = _pass_d(src, dst, xl2p, xr2p, sc2)
    return out2.reshape(NPT, 16)[:N, 0:1]


# speedup vs baseline: 4.9514x; 1.0655x over previous
import jax
import jax.numpy as jnp
from jax import lax
from jax.experimental import pallas as pl
from jax.experimental.pallas import tpu as pltpu
from jax.experimental.pallas import tpu_sc as plsc

N = 10000
E = 160000
H1, C1 = 8, 120
CP = 128
D = H1 * CP
W = 32
EPW = E // W
GB = 8
NB = EPW // GB
NPW = 320
NPT = W * NPW
CH = 1600
NCH = E // CH
RC = 125
NPC = N // RC
FL = 16

_MESH = dict(core_axis_name="c", subcore_axis_name="s")
_GDN = lax.GatherDimensionNumbers(
    offset_dims=(), collapsed_slice_dims=(0,), start_index_map=(0,))


def _lane_perm(x, idx):
    return lax.gather(x, idx[:, None], _GDN, (1,),
                      mode=lax.GatherScatterMode.PROMISE_IN_BOUNDS)


def _lane_iota():
    return lax.broadcasted_iota(jnp.int32, (16,), 0)


def _to_scalar(v):
    return jnp.where(_lane_iota() == 0, v, jnp.zeros_like(v))[0]


def _dyn_lane(v, i):
    return _to_scalar(_lane_perm(v, jnp.zeros((16,), jnp.int32) + i))


def _lane_count(mask):
    lane = _lane_iota()
    x = jnp.where(mask, jnp.ones((16,), jnp.int32), jnp.zeros((16,), jnp.int32))
    for sft in (1, 2, 4, 8):
        x = x + _lane_perm(x, (lane + sft) & 15)
    return x


def _lane_min(x):
    lane = _lane_iota()
    for sft in (1, 2, 4, 8):
        x = jnp.minimum(x, _lane_perm(x, (lane + sft) & 15))
    return x


def _wid():
    return lax.axis_index("s") * 2 + lax.axis_index("c")


def _proj_kernel(x_ref, wl_ref, wr_ref, bl_ref, br_ref, xl_ref, xr_ref):
    x = x_ref[...]
    xl_ref[...] = jnp.dot(x, wl_ref[...], preferred_element_type=jnp.float32) + bl_ref[...]
    xr_ref[...] = jnp.dot(x, wr_ref[...], preferred_element_type=jnp.float32) + br_ref[...]


def _proj(x, Wl, bl, Wr, br):
    n, f = x.shape
    k = Wl.shape[1]
    blk = 1000
    return pl.pallas_call(
        _proj_kernel,
        grid=(n // blk,),
        in_specs=[
            pl.BlockSpec((blk, f), lambda i: (i, 0)),
            pl.BlockSpec((f, k), lambda i: (0, 0)),
            pl.BlockSpec((f, k), lambda i: (0, 0)),
            pl.BlockSpec((k,), lambda i: (0,)),
            pl.BlockSpec((k,), lambda i: (0,)),
        ],
        out_specs=[
            pl.BlockSpec((blk, k), lambda i: (i, 0)),
            pl.BlockSpec((blk, k), lambda i: (i, 0)),
        ],
        out_shape=[
            jax.ShapeDtypeStruct((n, k), jnp.float32),
            jax.ShapeDtypeStruct((n, k), jnp.float32),
        ],
    )(x, Wl, Wr, bl, br)


def _pass_a_body(xl_hbm, xr_hbm, src_hbm, dst_hbm, att_hbm, alpha_hbm,
                 src_v, dst_v, att_v, lb0, lb1, rb0, rb1, a0, a1,
                 gs0, gs1, ws0, ws1):
    base = _wid() * EPW
    pltpu.sync_copy(src_hbm.at[pl.ds(base, EPW)], src_v)
    pltpu.sync_copy(dst_hbm.at[pl.ds(base, EPW)], dst_v)
    pltpu.sync_copy(att_hbm, att_v)
    zeros16 = jnp.zeros((16,), jnp.float32)
    lane = _lane_iota()
    rots = [(lane + s) & 15 for s in (1, 2, 4, 8)]

    def fire(k, lb, rb, gs):
        pltpu.async_copy(xl_hbm.at[src_v.at[pl.ds(k * GB, GB)]], lb, gs)
        pltpu.async_copy(xr_hbm.at[dst_v.at[pl.ds(k * GB, GB)]], rb, gs)

    def waitg(lb, rb, gs):
        pltpu.make_async_copy(xl_hbm.at[pl.ds(0, GB)], lb, gs).wait()
        pltpu.make_async_copy(xr_hbm.at[pl.ds(0, GB)], rb, gs).wait()

    def compute(k, lb, rb, ab, ws):
        @pl.when(k >= 2)
        def _():
            pltpu.make_async_copy(ab, alpha_hbm.at[pl.ds(base * 16, GB * 16)], ws).wait()

        def edge_body(e, _):
            row = zeros16
            for h in range(H1):
                acc = zeros16
                for v in range(CP // 16):
                    off = h * CP + v * 16
                    t = lb[e, pl.ds(off, 16)] + rb[e, pl.ds(off, 16)]
                    t = jnp.maximum(t, t * 0.2)
                    acc = acc + t * att_v[h, pl.ds(v * 16, 16)]
                for r in rots:
                    acc = acc + _lane_perm(acc, r)
                row = jnp.where(lane == h, acc, row)
            eo = pl.multiple_of(e * 16, 16)
            ab[pl.ds(eo, 16)] = row
            return 0

        lax.fori_loop(0, GB, edge_body, 0)
        pltpu.async_copy(ab, alpha_hbm.at[pl.ds((base + k * GB) * 16, GB * 16)], ws)

    fire(0, lb0, rb0, gs0)
    fire(1, lb1, rb1, gs1)

    def loop_body(bi, _):
        k0 = 2 * bi
        waitg(lb0, rb0, gs0)
        compute(k0, lb0, rb0, a0, ws0)

        @pl.when(k0 + 2 < NB)
        def _():
            fire(k0 + 2, lb0, rb0, gs0)

        waitg(lb1, rb1, gs1)
        compute(k0 + 1, lb1, rb1, a1, ws1)

        @pl.when(k0 + 3 < NB)
        def _():
            fire(k0 + 3, lb1, rb1, gs1)

        return 0

    lax.fori_loop(0, NB // 2, loop_body, 0)
    waitg(lb0, rb0, gs0)
    compute(NB - 1, lb0, rb0, a0, ws0)
    pltpu.make_async_copy(a0, alpha_hbm.at[pl.ds(base * 16, GB * 16)], ws0).wait()
    pltpu.make_async_copy(a1, alpha_hbm.at[pl.ds(base * 16, GB * 16)], ws1).wait()


def _pass_a(xlp, xrp, src, dst, attp):
    kfn = pl.kernel(
        _pass_a_body,
        out_type=jax.ShapeDtypeStruct((E * 16,), jnp.float32),
        mesh=plsc.VectorSubcoreMesh(**_MESH),
        scratch_types=[
            pltpu.VMEM((EPW,), jnp.int32),
            pltpu.VMEM((EPW,), jnp.int32),
            pltpu.VMEM((H1, CP), jnp.float32),
            pltpu.VMEM((GB, D), jnp.float32),
            pltpu.VMEM((GB, D), jnp.float32),
            pltpu.VMEM((GB, D), jnp.float32),
            pltpu.VMEM((GB, D), jnp.float32),
            pltpu.VMEM((GB * 16,), jnp.float32),
            pltpu.VMEM((GB * 16,), jnp.float32),
            pltpu.SemaphoreType.DMA,
            pltpu.SemaphoreType.DMA,
            pltpu.SemaphoreType.DMA,
            pltpu.SemaphoreType.DMA,
        ],
    )
    return kfn(xlp, xrp, src, dst, attp)


def _pass_b_body(dst_hbm, alpha_hbm, mtab_hbm, stab_hbm,
                 dv0, av0, dv1, av1, mtab, stab, cs0, cs1):
    lo = _wid() * NPW
    neg = jnp.full((16,), -1e30, jnp.float32)
    zeros16 = jnp.zeros((16,), jnp.float32)
    lane = _lane_iota()

    def init_body(i, _):
        o = pl.multiple_of(i * 16, 16)
        mtab[pl.ds(o, 16)] = neg
        stab[pl.ds(o, 16)] = zeros16
        return 0

    lax.fori_loop(0, NPW, init_body, 0)

    def fire(ci, dv, av, cs):
        pltpu.async_copy(dst_hbm.at[pl.ds(ci * CH, CH)], dv, cs)
        pltpu.async_copy(alpha_hbm.at[pl.ds(ci * CH * 16, CH * 16)], av, cs)

    def waitc(dv, av, cs):
        pltpu.make_async_copy(dst_hbm.at[pl.ds(0, CH)], dv, cs).wait()
        pltpu.make_async_copy(alpha_hbm.at[pl.ds(0, CH * 16)], av, cs).wait()

    def process(dv, av):
        def group(gi, _):
            gbase = pl.multiple_of(gi * 16, 16)
            d = dv[pl.ds(gbase, 16)]
            dl = d - lo
            mask = (dl >= 0) & (dl < NPW)
            tot = _lane_count(mask)
            lv0 = jnp.where(mask, lane, 16)

            def wbody(k, lv):
                i0v = _lane_min(lv) & 15
                i0 = _to_scalar(i0v)
                dli = _dyn_lane(dl, i0)
                to = pl.multiple_of(dli * 16, 16)
                ao = pl.multiple_of((gbase + i0) * 16, 16)
                arow = av[pl.ds(ao, 16)]
                m0 = mtab[pl.ds(to, 16)]
                mn = jnp.maximum(m0, arow)
                em = jnp.exp(m0 - mn)
                stab[pl.ds(to, 16)] = stab[pl.ds(to, 16)] * em + jnp.exp(arow - mn)
                mtab[pl.ds(to, 16)] = mn
                return jnp.where(lane == i0v, 16, lv)

            lax.fori_loop(0, _to_scalar(tot), wbody, lv0)
            return 0

        lax.fori_loop(0, CH // 16, group, 0)

    fire(0, dv0, av0, cs0)
    fire(1, dv1, av1, cs1)

    def chunk_loop(hi, _):
        c0 = 2 * hi
        waitc(dv0, av0, cs0)
        process(dv0, av0)

        @pl.when(c0 + 2 < NCH)
        def _():
            fire(c0 + 2, dv0, av0, cs0)

        waitc(dv1, av1, cs1)
        process(dv1, av1)

        @pl.when(c0 + 3 < NCH)
        def _():
            fire(c0 + 3, dv1, av1, cs1)

        return 0

    lax.fori_loop(0, NCH // 2, chunk_loop, 0)
    pltpu.sync_copy(mtab, mtab_hbm.at[pl.ds(lo * 16, NPW * 16)])
    pltpu.sync_copy(stab, stab_hbm.at[pl.ds(lo * 16, NPW * 16)])


def _pass_b(dst, alpha):
    kfn = pl.kernel(
        _pass_b_body,
        out_type=[
            jax.ShapeDtypeStruct((NPT * 16,), jnp.float32),
            jax.ShapeDtypeStruct((NPT * 16,), jnp.float32),
        ],
        mesh=plsc.VectorSubcoreMesh(**_MESH),
        scratch_types=[
            pltpu.VMEM((CH,), jnp.int32),
            pltpu.VMEM((CH * 16,), jnp.float32),
            pltpu.VMEM((CH,), jnp.int32),
            pltpu.VMEM((CH * 16,), jnp.float32),
            pltpu.VMEM((NPW * 16,), jnp.float32),
            pltpu.VMEM((NPW * 16,), jnp.float32),
            pltpu.SemaphoreType.DMA,
            pltpu.SemaphoreType.DMA,
        ],
    )
    return kfn(dst, alpha)


def _pass_c_body(xl_hbm, src_hbm, dst_hbm, alpha_hbm, mtab_hbm, stab_hbm,
                 bias_hbm, hout_hbm,
                 sv0, dv0, sv1, dv1, mtv, stv, acc, biasv,
                 idxs, idxw, xst, astw, pdref, psref, pjref, pcref,
                 fdref, fcref, cs0, cs1, gs0):
    wid = _wid()
    pltpu.sync_copy(bias_hbm, biasv)
    zeros16 = jnp.zeros((16,), jnp.float32)
    zi = jnp.zeros((16,), jnp.int32)
    lane = _lane_iota()

    def flush_fire(pd, ps, pj):
        idxs[...] = ps

        def widx(e, _):
            o = pl.multiple_of(e * 16, 16)
            pjv = _lane_perm(pj, jnp.zeros((16,), jnp.int32) + e)
            idxw[pl.ds(o, 16)] = pjv * 16 + lane
            return 0

        lax.fori_loop(0, FL, widx, 0)
        pltpu.async_copy(xl_hbm.at[idxs], xst, gs0)
        pltpu.async_copy(alpha_hbm.at[idxw], astw, gs0)

    def flush_drain(cnt, pd):
        pltpu.make_async_copy(xl_hbm.at[pl.ds(0, FL)], xst, gs0).wait()
        pltpu.make_async_copy(alpha_hbm.at[idxw], astw, gs0).wait()

        def pe(e, _):
            dl_e = _dyn_lane(pd, e)
            to = pl.multiple_of(dl_e * 16, 16)
            ao = pl.multiple_of(e * 16, 16)
            arow = astw[pl.ds(ao, 16)]
            c = jnp.exp(arow - mtv[pl.ds(to, 16)]) / (stv[pl.ds(to, 16)] + 1e-16)
            for h in range(H1):
                chv = _lane_perm(c, jnp.full((16,), h, jnp.int32))
                for v in range(CP // 16):
                    off = h * CP + v * 16
                    plsc.addupdate(acc.at[dl_e, pl.ds(off, 16)],
                                   xst[e, pl.ds(off, 16)] * chv)
            return 0

        lax.fori_loop(0, cnt, pe, 0)

    def range_pass(rp, _):
        rng = rp * W + wid

        @pl.when(rng < RC)
        def _():
            lo = rng * NPC
            pltpu.sync_copy(mtab_hbm.at[pl.ds(lo * 16, NPC * 16)], mtv)
            pltpu.sync_copy(stab_hbm.at[pl.ds(lo * 16, NPC * 16)], stv)

            def z(i, _):
                for v in range(D // 16):
                    acc[i, pl.ds(v * 16, 16)] = zeros16
                return 0

            lax.fori_loop(0, NPC, z, 0)

            def do_chunk(c0, sv, dv):
                def subgroup(gbase, dl, sg, mask):
                    tot = _lane_count(mask)
                    tot_s = _to_scalar(tot)

                    @pl.when(tot_s > 0)
                    def _():
                        pd = pdref[pl.ds(0, 16)]
                        ps = psref[pl.ds(0, 16)]
                        pj = pjref[pl.ds(0, 16)]
                        pcv = pcref[pl.ds(0, 16)]
                        lv0 = jnp.where(mask, lane, 16)

                        def compact(k, st4):
                            hd, hs, hj, lv = st4
                            i0v = _lane_min(lv) & 15
                            hd = jnp.where(lane == k, _lane_perm(dl, i0v), hd)
                            hs = jnp.where(lane == k, _lane_perm(sg, i0v), hs)
                            hj = jnp.where(lane == k, c0 * CH + gbase + i0v, hj)
                            lv = jnp.where(lane == i0v, 16, lv)
                            return (hd, hs, hj, lv)

                        hd, hs, hj, _lv = lax.fori_loop(
                            0, tot_s, compact, (zi, zi, zi, lv0))
                        shd = _lane_perm(hd, (lane - pcv) & 15)
                        shs = _lane_perm(hs, (lane - pcv) & 15)
                        shj = _lane_perm(hj, (lane - pcv) & 15)
                        pd_n = jnp.where(lane >= pcv, shd, pd)
                        ps_n = jnp.where(lane >= pcv, shs, ps)
                        pj_n = jnp.where(lane >= pcv, shj, pj)
                        mv = pcv + tot

                        @pl.when(_to_scalar(mv) >= FL)
                        def _():
                            fc = _to_scalar(fcref[pl.ds(0, 16)])

                            @pl.when(fc > 0)
                            def _():
                                flush_drain(FL, fdref[pl.ds(0, 16)])

                            flush_fire(pd_n, ps_n, pj_n)
                            fdref[...] = pd_n
                            fcref[...] = jnp.zeros((16,), jnp.int32) + FL

                        pd_a = _lane_perm(hd, (lane + FL - pcv) & 15)
                        ps_a = _lane_perm(hs, (lane + FL - pcv) & 15)
                        pj_a = _lane_perm(hj, (lane + FL - pcv) & 15)
                        ovf = mv >= FL
                        pdref[...] = jnp.where(ovf, pd_a, pd_n)
                        psref[...] = jnp.where(ovf, ps_a, ps_n)
                        pjref[...] = jnp.where(ovf, pj_a, pj_n)
                        pcref[...] = jnp.where(ovf, mv - FL, mv)

                def group(gi, _):
                    gb = pl.multiple_of(gi * 32, 16)
                    d1 = dv[pl.ds(gb, 16)]
                    d2 = dv[pl.ds(gb + 16, 16)]
                    dl1 = d1 - lo
                    dl2 = d2 - lo
                    mask1 = (dl1 >= 0) & (dl1 < NPC)
                    mask2 = (dl2 >= 0) & (dl2 < NPC)
                    lv1 = jnp.where(mask1, lane, 16)
                    lv2 = jnp.where(mask2, lane, 16)
                    gate = _to_scalar(_lane_min(jnp.minimum(lv1, lv2)))

                    @pl.when(gate < 16)
                    def _():
                        subgroup(gb, dl1, sv[pl.ds(gb, 16)], mask1)
                        subgroup(gb + 16, dl2, sv[pl.ds(gb + 16, 16)], mask2)

                    return 0

                lax.fori_loop(0, CH // 32, group, 0)

            def fire(ci, sv, dv, cs):
                pltpu.async_copy(src_hbm.at[pl.ds(ci * CH, CH)], sv, cs)
                pltpu.async_copy(dst_hbm.at[pl.ds(ci * CH, CH)], dv, cs)

            def waitc(sv, dv, cs):
                pltpu.make_async_copy(src_hbm.at[pl.ds(0, CH)], sv, cs).wait()
                pltpu.make_async_copy(dst_hbm.at[pl.ds(0, CH)], dv, cs).wait()

            pdref[...] = zi
            psref[...] = zi
            pjref[...] = zi
            pcref[...] = zi
            fcref[...] = zi
            fire(0, sv0, dv0, cs0)
            fire(1, sv1, dv1, cs1)

            def chunk_loop(hi, _):
                c0 = 2 * hi
                waitc(sv0, dv0, cs0)
                do_chunk(c0, sv0, dv0)

                @pl.when(c0 + 2 < NCH)
                def _():
                    fire(c0 + 2, sv0, dv0, cs0)

                waitc(sv1, dv1, cs1)
                do_chunk(c0 + 1, sv1, dv1)

                @pl.when(c0 + 3 < NCH)
                def _():
                    fire(c0 + 3, sv1, dv1, cs1)

                return 0

            lax.fori_loop(0, NCH // 2, chunk_loop, 0)
            fc = _to_scalar(fcref[pl.ds(0, 16)])

            @pl.when(fc > 0)
            def _():
                flush_drain(FL, fdref[pl.ds(0, 16)])

            pc = _to_scalar(pcref[pl.ds(0, 16)])

            @pl.when(pc > 0)
            def _():
                flush_fire(pdref[pl.ds(0, 16)], psref[pl.ds(0, 16)],
                           pjref[pl.ds(0, 16)])
                flush_drain(pc, pdref[pl.ds(0, 16)])

            def ep(i, _):
                for v in range(D // 16):
                    o = v * 16
                    acc[i, pl.ds(o, 16)] = jnp.maximum(
                        acc[i, pl.ds(o, 16)] + biasv[pl.ds(o, 16)], 0.0)
                return 0

            lax.fori_loop(0, NPC, ep, 0)
            pltpu.sync_copy(acc, hout_hbm.at[pl.ds(lo, NPC)])

        return 0

    lax.fori_loop(0, (RC + W - 1) // W, range_pass, 0)


def _pass_c(xlp, src, dst, alpha, mtab, stab, biasp):
    kfn = pl.kernel(
        _pass_c_body,
        out_type=jax.ShapeDtypeStruct((N, D), jnp.float32),
        mesh=plsc.VectorSubcoreMesh(**_MESH),
        scratch_types=[
            pltpu.VMEM((CH,), jnp.int32),
            pltpu.VMEM((CH,), jnp.int32),
            pltpu.VMEM((CH,), jnp.int32),
            pltpu.VMEM((CH,), jnp.int32),
            pltpu.VMEM((NPC * 16,), jnp.float32),
            pltpu.VMEM((NPC * 16,), jnp.float32),
            pltpu.VMEM((NPC, D), jnp.float32),
            pltpu.VMEM((D,), jnp.float32),
            pltpu.VMEM((FL,), jnp.int32),
            pltpu.VMEM((FL * 16,), jnp.int32),
            pltpu.VMEM((FL, D), jnp.float32),
            pltpu.VMEM((FL * 16,), jnp.float32),
            pltpu.VMEM((16,), jnp.int32),
            pltpu.VMEM((16,), jnp.int32),
            pltpu.VMEM((16,), jnp.int32),
            pltpu.VMEM((16,), jnp.int32),
            pltpu.VMEM((16,), jnp.int32),
            pltpu.VMEM((16,), jnp.int32),
            pltpu.SemaphoreType.DMA,
            pltpu.SemaphoreType.DMA,
            pltpu.SemaphoreType.DMA,
        ],
    )
    return kfn(xlp, src, dst, alpha, mtab, stab, biasp)


def _pass_d_body(src_hbm, dst_hbm, xl2_hbm, xr2_hbm, sc2_hbm, out_hbm,
                 sv0, dv0, sv1, dv1, xl2v, xr2v, mt, st, vt, sc2v, cs0, cs1):
    lo = _wid() * NPW
    neg = jnp.full((16,), -1e30, jnp.float32)
    zeros16 = jnp.zeros((16,), jnp.float32)
    lane = _lane_iota()
    pltpu.sync_copy(xl2_hbm, xl2v)
    pltpu.sync_copy(xr2_hbm, xr2v)
    pltpu.sync_copy(sc2_hbm, sc2v)
    scv = sc2v[pl.ds(0, 16)]
    att2s = scv[0]
    bias2s = scv[1]

    def init_body(i, _):
        o = pl.multiple_of(i * 16, 16)
        mt[pl.ds(o, 16)] = neg
        st[pl.ds(o, 16)] = zeros16
        vt[pl.ds(o, 16)] = zeros16
        return 0

    lax.fori_loop(0, NPW, init_body, 0)

    def fire(ci, sv, dv, cs):
        pltpu.async_copy(src_hbm.at[pl.ds(ci * CH, CH)], sv, cs)
        pltpu.async_copy(dst_hbm.at[pl.ds(ci * CH, CH)], dv, cs)

    def waitc(sv, dv, cs):
        pltpu.make_async_copy(src_hbm.at[pl.ds(0, CH)], sv, cs).wait()
        pltpu.make_async_copy(dst_hbm.at[pl.ds(0, CH)], dv, cs).wait()

    def tab_read(tab, i):
        ib = pl.multiple_of((i >> 4) << 4, 16)
        return _dyn_lane(tab[pl.ds(ib, 16)], i & 15)

    def process(sv, dv):
        def group(gi, _):
            gbase = pl.multiple_of(gi * 16, 16)
            d = dv[pl.ds(gbase, 16)]
            sg = sv[pl.ds(gbase, 16)]
            dl = d - lo
            mask = (dl >= 0) & (dl < NPW)
            tot = _lane_count(mask)
            lv0 = jnp.where(mask, lane, 16)

            def wbody(k, lv):
                i0 = _to_scalar(_lane_min(lv))
                dli = _dyn_lane(dl, i0)
                si = _dyn_lane(sg, i0)
                di = _dyn_lane(d, i0)
                xls = tab_read(xl2v, si)
                t = xls + tab_read(xr2v, di)
                a2 = att2s * jnp.maximum(t, 0.2 * t)
                to = pl.multiple_of(dli * 16, 16)
                m0 = mt[pl.ds(to, 16)]
                mn = jnp.maximum(m0, a2)
                em = jnp.exp(m0 - mn)
                ea = jnp.exp(a2 - mn)
                st[pl.ds(to, 16)] = st[pl.ds(to, 16)] * em + ea
                vt[pl.ds(to, 16)] = vt[pl.ds(to, 16)] * em + ea * xls
                mt[pl.ds(to, 16)] = mn
                return jnp.where(lane == i0, 16, lv)

            lax.fori_loop(0, _to_scalar(tot), wbody, lv0)
            return 0

        lax.fori_loop(0, CH // 16, group, 0)

    fire(0, sv0, dv0, cs0)
    fire(1, sv1, dv1, cs1)

    def chunk_loop(hi, _):
        c0 = 2 * hi
        waitc(sv0, dv0, cs0)
        process(sv0, dv0)

        @pl.when(c0 + 2 < NCH)
        def _():
            fire(c0 + 2, sv0, dv0, cs0)

        waitc(sv1, dv1, cs1)
        process(sv1, dv1)

        @pl.when(c0 + 3 < NCH)
        def _():
            fire(c0 + 3, sv1, dv1, cs1)

        return 0

    lax.fori_loop(0, NCH // 2, chunk_loop, 0)

    def ep(i, _):
        o = pl.multiple_of(i * 16, 16)
        vt[pl.ds(o, 16)] = vt[pl.ds(o, 16)] / (st[pl.ds(o, 16)] + 1e-16) + bias2s
        return 0

    lax.fori_loop(0, NPW, ep, 0)
    pltpu.sync_copy(vt, out_hbm.at[pl.ds(lo * 16, NPW * 16)])


def _pass_d(src, dst, xl2, xr2, sc2):
    kfn = pl.kernel(
        _pass_d_body,
        out_type=jax.ShapeDtypeStruct((NPT * 16,), jnp.float32),
        mesh=plsc.VectorSubcoreMesh(**_MESH),
        scratch_types=[
            pltpu.VMEM((CH,), jnp.int32),
            pltpu.VMEM((CH,), jnp.int32),
            pltpu.VMEM((CH,), jnp.int32),
            pltpu.VMEM((CH,), jnp.int32),
            pltpu.VMEM((NPT,), jnp.float32),
            pltpu.VMEM((NPT,), jnp.float32),
            pltpu.VMEM((NPW * 16,), jnp.float32),
            pltpu.VMEM((NPW * 16,), jnp.float32),
            pltpu.VMEM((NPW * 16,), jnp.float32),
            pltpu.VMEM((16,), jnp.float32),
            pltpu.SemaphoreType.DMA,
            pltpu.SemaphoreType.DMA,
        ],
    )
    return kfn(src, dst, xl2, xr2, sc2)


def _pad_heads(w, heads, ch):
    lead = w.shape[:-1]
    w = w.reshape(lead + (heads, ch))
    w = jnp.pad(w, [(0, 0)] * len(lead) + [(0, 0), (0, CP - ch)])
    return w.reshape(lead + (heads * CP,))


def kernel(x, edge_index, Wl1, bl1, Wr1, br1, att1, bias1, Wl2, bl2, Wr2, br2, att2, bias2):
    src = edge_index[0]
    dst = edge_index[1]

    Wl1p = _pad_heads(Wl1, H1, C1)
    Wr1p = _pad_heads(Wr1, H1, C1)
    bl1p = _pad_heads(bl1, H1, C1)
    br1p = _pad_heads(br1, H1, C1)
    attp = _pad_heads(att1.reshape(1, H1 * C1), H1, C1).reshape(H1, CP)
    bias1p = _pad_heads(bias1, H1, C1)

    xlp, xrp = _proj(x, Wl1p, bl1p, Wr1p, br1p)
    alpha = _pass_a(xlp, xrp, src, dst, attp)
    mtab, stab = _pass_b(dst, alpha)
    hp = _pass_c(xlp, src, dst, alpha, mtab, stab, bias1p)

    Wl2p = _pad_heads(Wl2.T, H1, C1).T
    Wr2p = _pad_heads(Wr2.T, H1, C1).T
    xl2, xr2 = _proj(hp, Wl2p, bl2, Wr2p, br2)
    xl2p = jnp.pad(xl2[:, 0], (0, NPT - N))
    xr2p = jnp.pad(xr2[:, 0], (0, NPT - N))
    sc2 = jnp.concatenate([att2.reshape(1), bias2.reshape(1),
                           jnp.zeros((14,), jnp.float32)])
    out2 = _pass_d(src, dst, xl2p, xr2p, sc2)
    return out2.reshape(NPT, 16)[:N, 0:1]
